# Initial kernel scaffold; baseline (speedup 1.0000x reference)
import jax
import jax.numpy as jnp
from jax.experimental import pallas as pl


def kernel(nodes, edges, senders, receivers, We1, be1, We2, be2, We3, be3, ge, bge, Wn1, bn1, Wn2, bn2, Wn3, bn3, gn, bgn):
    raise NotImplementedError("write your pallas kernel here")


# speedup vs baseline: 1.9677x; 1.9677x over previous
import functools

import jax
import jax.numpy as jnp
from jax import lax
from jax.experimental import pallas as pl
from jax.experimental.pallas import tpu as pltpu
from jax.experimental.pallas import tpu_sc as plsc

N, E, H = 10000, 320000, 128
NC, NS = 2, 16
NW = NC * NS
EPW = E // NW
CH = 80
NCHUNK = EPW // CH

@functools.cache
def _mesh():
    return plsc.VectorSubcoreMesh(core_axis_name="c", subcore_axis_name="s",
                                  num_cores=NC, num_subcores=NS)


def _gather_body(nodes_hbm, send_hbm, recv_hbm, src_out, dst_out,
                 idx_v, rows_v, sem):
    c = lax.axis_index("c")
    s = lax.axis_index("s")
    base = (c * NS + s) * EPW

    def phase(idx_hbm, out_hbm):
        def chunk(j, carry):
            off = base + j * CH
            pltpu.sync_copy(idx_hbm.at[pl.ds(off, CH)], idx_v)
            pltpu.async_copy(nodes_hbm.at[idx_v], rows_v, sem).wait()
            pltpu.sync_copy(rows_v, out_hbm.at[pl.ds(off, CH)])
            return carry
        lax.fori_loop(0, NCHUNK, chunk, 0)

    phase(send_hbm, src_out)
    phase(recv_hbm, dst_out)


@jax.jit
def _gather(nodes, senders, receivers):
    return pl.kernel(
        _gather_body,
        out_type=(jax.ShapeDtypeStruct((E, H), jnp.float32),
                  jax.ShapeDtypeStruct((E, H), jnp.float32)),
        mesh=_mesh(),
        scratch_types=[
            pltpu.VMEM((CH,), jnp.int32),
            pltpu.VMEM((CH, H), jnp.float32),
            pltpu.SemaphoreType.DMA,
        ],
    )(nodes, senders, receivers)


def _scatter_body(eout_hbm, recv_hbm, zeros_hbm, acc_out,
                  idx_v, rows_v, eff_sh):
    c = lax.axis_index("c")
    s = lax.axis_index("s")
    base = (c * NS + s) * EPW
    rpw = 624
    tail_off = rpw * NS
    tail = N - tail_off

    pltpu.sync_copy(zeros_hbm.at[pl.ds(s * rpw, rpw)],
                    eff_sh.at[pl.ds(s * rpw, rpw)])
    @pl.when(s == 0)
    def _():
        pltpu.sync_copy(zeros_hbm.at[pl.ds(tail_off, tail)],
                        eff_sh.at[pl.ds(tail_off, tail)])
    plsc.subcore_barrier()

    def chunk(j, carry):
        off = base + j * CH
        pltpu.sync_copy(recv_hbm.at[pl.ds(off, CH)], idx_v)
        pltpu.sync_copy(eout_hbm.at[pl.ds(off, CH)], rows_v)
        pltpu.sync_copy(rows_v, eff_sh.at[idx_v], add=True)
        return carry
    lax.fori_loop(0, NCHUNK, chunk, 0)
    plsc.subcore_barrier()

    pltpu.sync_copy(eff_sh.at[pl.ds(s * rpw, rpw)],
                    acc_out.at[c, pl.ds(s * rpw, rpw)])
    @pl.when(s == 0)
    def _():
        pltpu.sync_copy(eff_sh.at[pl.ds(tail_off, tail)],
                        acc_out.at[c, pl.ds(tail_off, tail)])


@jax.jit
def _scatter(edges_out, receivers, zeros):
    return pl.kernel(
        _scatter_body,
        out_type=jax.ShapeDtypeStruct((NC, N, H), jnp.float32),
        mesh=_mesh(),
        scratch_types=[
            pltpu.VMEM((CH,), jnp.int32),
            pltpu.VMEM((CH, H), jnp.float32),
            pltpu.VMEM_SHARED((N, H), jnp.float32),
        ],
    )(edges_out, receivers, zeros)


def _edge_mlp_body(dst_ref, src_ref, edg_ref, wa_ref, wb_ref, wc_ref,
                   b1_ref, w2_ref, b2_ref, w3_ref, b3_ref, g_ref, bg_ref,
                   out_ref):
    f32 = jnp.float32
    h = jnp.dot(dst_ref[...], wa_ref[...], preferred_element_type=f32)
    h += jnp.dot(src_ref[...], wb_ref[...], preferred_element_type=f32)
    h += jnp.dot(edg_ref[...], wc_ref[...], preferred_element_type=f32)
    h = jnp.maximum(h + b1_ref[...], 0.0)
    h = jnp.maximum(
        jnp.dot(h, w2_ref[...], preferred_element_type=f32) + b2_ref[...], 0.0)
    h = jnp.dot(h, w3_ref[...], preferred_element_type=f32) + b3_ref[...]
    mu = jnp.mean(h, axis=-1, keepdims=True)
    d = h - mu
    var = jnp.mean(d * d, axis=-1, keepdims=True)
    out_ref[...] = d * lax.rsqrt(var + 1e-5) * g_ref[...] + bg_ref[...]


BE = 512


@jax.jit
def _edge_mlp(dst, src, edg, wa, wb, wc, b1, w2, b2, w3, b3, g, bg):
    row_spec = pl.BlockSpec((BE, H), lambda i: (i, 0))
    w_spec = pl.BlockSpec((H, H), lambda i: (0, 0))
    v_spec = pl.BlockSpec((1, H), lambda i: (0, 0))
    return pl.pallas_call(
        _edge_mlp_body,
        grid=(E // BE,),
        in_specs=[row_spec, row_spec, row_spec,
                  w_spec, w_spec, w_spec, v_spec,
                  w_spec, v_spec, w_spec, v_spec, v_spec, v_spec],
        out_specs=row_spec,
        out_shape=jax.ShapeDtypeStruct((E, H), jnp.float32),
    )(dst, src, edg, wa, wb, wc, b1, w2, b2, w3, b3, g, bg)


def _node_mlp_body(nod_ref, acc_ref, wa_ref, wb_ref,
                   b1_ref, w2_ref, b2_ref, w3_ref, b3_ref, g_ref, bg_ref,
                   out_ref):
    f32 = jnp.float32
    eff = acc_ref[0] + acc_ref[1]
    h = jnp.dot(nod_ref[...], wa_ref[...], preferred_element_type=f32)
    h += jnp.dot(eff, wb_ref[...], preferred_element_type=f32)
    h = jnp.maximum(h + b1_ref[...], 0.0)
    h = jnp.maximum(
        jnp.dot(h, w2_ref[...], preferred_element_type=f32) + b2_ref[...], 0.0)
    h = jnp.dot(h, w3_ref[...], preferred_element_type=f32) + b3_ref[...]
    mu = jnp.mean(h, axis=-1, keepdims=True)
    d = h - mu
    var = jnp.mean(d * d, axis=-1, keepdims=True)
    out_ref[...] = d * lax.rsqrt(var + 1e-5) * g_ref[...] + bg_ref[...]


BN = 1000


@jax.jit
def _node_mlp(nodes, acc, wa, wb, b1, w2, b2, w3, b3, g, bg):
    row_spec = pl.BlockSpec((BN, H), lambda i: (i, 0))
    acc_spec = pl.BlockSpec((NC, BN, H), lambda i: (0, i, 0))
    w_spec = pl.BlockSpec((H, H), lambda i: (0, 0))
    v_spec = pl.BlockSpec((1, H), lambda i: (0, 0))
    return pl.pallas_call(
        _node_mlp_body,
        grid=(N // BN,),
        in_specs=[row_spec, acc_spec,
                  w_spec, w_spec, v_spec,
                  w_spec, v_spec, w_spec, v_spec, v_spec, v_spec],
        out_specs=row_spec,
        out_shape=jax.ShapeDtypeStruct((N, H), jnp.float32),
    )(nodes, acc, wa, wb, b1, w2, b2, w3, b3, g, bg)


def kernel(nodes, edges, senders, receivers, We1, be1, We2, be2, We3, be3,
           ge, bge, Wn1, bn1, Wn2, bn2, Wn3, bn3, gn, bgn):
    senders = senders.astype(jnp.int32)
    receivers = receivers.astype(jnp.int32)
    r1 = lambda v: v.reshape(1, H)

    src, dst = _gather(nodes, senders, receivers)
    edges_out = _edge_mlp(
        dst, src, edges,
        We1[:H], We1[H:2 * H], We1[2 * H:], r1(be1),
        We2, r1(be2), We3, r1(be3), r1(ge), r1(bge))
    zeros = jnp.zeros((N, H), jnp.float32)
    acc = _scatter(edges_out, receivers, zeros)
    nodes_out = _node_mlp(
        nodes, acc,
        Wn1[:H], Wn1[H:], r1(bn1),
        Wn2, r1(bn2), Wn3, r1(bn3), r1(gn), r1(bgn))
    return (nodes_out, edges_out)


# speedup vs baseline: 2.4597x; 1.2500x over previous
import functools

import jax
import jax.numpy as jnp
from jax import lax
from jax.experimental import pallas as pl
from jax.experimental.pallas import tpu as pltpu
from jax.experimental.pallas import tpu_sc as plsc

N, E, H = 10000, 320000, 128
NC, NS = 2, 16
NW = NC * NS
EPW = E // NW
CH = 80
NCH = EPW // CH
NBUF = 5
HALF = N // NC
ACCR = HALF + 8
EPT = E // NS
NCHT = EPT // CH


@functools.cache
def _mesh():
    return plsc.VectorSubcoreMesh(core_axis_name="c", subcore_axis_name="s",
                                  num_cores=NC, num_subcores=NS)


def _worker_id():
    return lax.axis_index("c") * NS + lax.axis_index("s")


def _gather_body(nodes_hbm, sr_hbm, gout, idx_all, rows, gsems, wsems):
    wid = _worker_id()
    ebase = wid * EPW

    for k in range(2):
        pltpu.sync_copy(sr_hbm.at[k, wid], idx_all.at[k])

    def chunk_refs(j):
        sel = j // NCH
        r = j % NCH
        idx = idx_all.at[sel, r]
        out = gout.at[sel, pl.ds(ebase + r * CH, CH)]
        return idx, out

    for b in range(NBUF):
        idx, _ = chunk_refs(b)
        pltpu.async_copy(nodes_hbm.at[idx], rows[b], gsems[b])

    def group(jj, carry):
        for b in range(NBUF):
            j = jj * NBUF + b
            idx, out = chunk_refs(j)
            pltpu.make_async_copy(nodes_hbm.at[idx], rows[b], gsems[b]).wait()
            pltpu.async_copy(rows[b], out, wsems[b])
        for b in range(NBUF):
            j2 = (jj + 1) * NBUF + b
            @pl.when(j2 < 2 * NCH)
            def _():
                idx2, out2 = chunk_refs(j2)
                pltpu.make_async_copy(rows[b], out2, wsems[b]).wait()
                pltpu.async_copy(nodes_hbm.at[idx2], rows[b], gsems[b])
        return carry

    lax.fori_loop(0, (2 * NCH) // NBUF, group, 0)

    for b in range(NBUF):
        pltpu.make_async_copy(
            rows[b], gout.at[0, pl.ds(ebase, CH)], wsems[b]).wait()


@jax.jit
def _gather(nodes, sr3d):
    return pl.kernel(
        _gather_body,
        out_type=jax.ShapeDtypeStruct((2, E, H), jnp.float32),
        mesh=_mesh(),
        scratch_types=[
            pltpu.VMEM((2, NCH, CH), jnp.int32),
            [pltpu.VMEM((CH, H), jnp.float32) for _ in range(NBUF)],
            [pltpu.SemaphoreType.DMA for _ in range(NBUF)],
            [pltpu.SemaphoreType.DMA for _ in range(NBUF)],
        ],
    )(nodes, sr3d)


def _scatter_body(eout_hbm, recv_hbm, zeros_hbm, acc_out,
                  idx_all, rows, lsems, eff_sh):
    c = lax.axis_index("c")
    s = lax.axis_index("s")
    ebase = s * EPT
    lo = c * HALF

    pltpu.sync_copy(recv_hbm.at[s], idx_all)

    def remap(i, carry):
        r = i // (CH // 16)
        k = i % (CH // 16)
        v = idx_all[r, pl.ds(k * 16, 16)]
        v = v - lo
        keep = (v >= 0) & (v < HALF)
        idx_all[r, pl.ds(k * 16, 16)] = jnp.where(keep, v, HALF)
        return carry
    lax.fori_loop(0, NCHT * (CH // 16), remap, 0)

    rpz = 312
    tailz_off = rpz * NS
    tailz = ACCR - tailz_off
    pltpu.sync_copy(zeros_hbm.at[pl.ds(s * rpz, rpz)],
                    eff_sh.at[pl.ds(s * rpz, rpz)])
    @pl.when(s == 0)
    def _():
        pltpu.sync_copy(zeros_hbm.at[pl.ds(tailz_off, tailz)],
                        eff_sh.at[pl.ds(tailz_off, tailz)])
    plsc.subcore_barrier()

    def load_ref(j):
        return eout_hbm.at[pl.ds(ebase + j * CH, CH)]

    for b in range(NBUF):
        pltpu.async_copy(load_ref(b), rows[b], lsems[b])

    def group(jj, carry):
        for b in range(NBUF):
            j = jj * NBUF + b
            pltpu.make_async_copy(load_ref(j), rows[b], lsems[b]).wait()
            pltpu.sync_copy(rows[b], eff_sh.at[idx_all.at[j]], add=True)
            j2 = j + NBUF
            @pl.when(j2 < NCHT)
            def _():
                pltpu.async_copy(load_ref(j2), rows[b], lsems[b])
        return carry

    lax.fori_loop(0, NCHT // NBUF, group, 0)
    plsc.subcore_barrier()

    pltpu.sync_copy(eff_sh.at[pl.ds(s * rpz, rpz)],
                    acc_out.at[pl.ds(lo + s * rpz, rpz)])
    @pl.when(s == 0)
    def _():
        pltpu.sync_copy(eff_sh.at[pl.ds(tailz_off, HALF - tailz_off)],
                        acc_out.at[pl.ds(lo + tailz_off, HALF - tailz_off)])


@jax.jit
def _scatter(edges_out, recv3d, zeros):
    return pl.kernel(
        _scatter_body,
        out_type=jax.ShapeDtypeStruct((N, H), jnp.float32),
        mesh=_mesh(),
        scratch_types=[
            pltpu.VMEM((NCHT, CH), jnp.int32),
            [pltpu.VMEM((CH, H), jnp.float32) for _ in range(NBUF)],
            [pltpu.SemaphoreType.DMA for _ in range(NBUF)],
            pltpu.VMEM_SHARED((ACCR, H), jnp.float32),
        ],
    )(edges_out, recv3d, zeros)


def _edge_mlp_body(gath_ref_src, gath_ref_dst, edg_ref, wa_ref, wb_ref,
                   wc_ref, b1_ref, w2_ref, b2_ref, w3_ref, b3_ref, g_ref,
                   bg_ref, out_ref):
    f32 = jnp.float32
    h = jnp.dot(gath_ref_dst[0], wa_ref[...], preferred_element_type=f32)
    h += jnp.dot(gath_ref_src[0], wb_ref[...], preferred_element_type=f32)
    h += jnp.dot(edg_ref[...], wc_ref[...], preferred_element_type=f32)
    h = jnp.maximum(h + b1_ref[...], 0.0)
    h = jnp.maximum(
        jnp.dot(h, w2_ref[...], preferred_element_type=f32) + b2_ref[...], 0.0)
    h = jnp.dot(h, w3_ref[...], preferred_element_type=f32) + b3_ref[...]
    mu = jnp.mean(h, axis=-1, keepdims=True)
    d = h - mu
    var = jnp.mean(d * d, axis=-1, keepdims=True)
    out_ref[...] = d * lax.rsqrt(var + 1e-5) * g_ref[...] + bg_ref[...]


BE = 512


@jax.jit
def _edge_mlp(gath, edg, wa, wb, wc, b1, w2, b2, w3, b3, g, bg):
    src_spec = pl.BlockSpec((1, BE, H), lambda i: (0, i, 0))
    dst_spec = pl.BlockSpec((1, BE, H), lambda i: (1, i, 0))
    row_spec = pl.BlockSpec((BE, H), lambda i: (i, 0))
    w_spec = pl.BlockSpec((H, H), lambda i: (0, 0))
    v_spec = pl.BlockSpec((1, H), lambda i: (0, 0))
    return pl.pallas_call(
        _edge_mlp_body,
        grid=(E // BE,),
        in_specs=[src_spec, dst_spec, row_spec,
                  w_spec, w_spec, w_spec, v_spec,
                  w_spec, v_spec, w_spec, v_spec, v_spec, v_spec],
        out_specs=row_spec,
        out_shape=jax.ShapeDtypeStruct((E, H), jnp.float32),
    )(gath, gath, edg, wa, wb, wc, b1, w2, b2, w3, b3, g, bg)


def _node_mlp_body(nod_ref, acc_ref, wa_ref, wb_ref,
                   b1_ref, w2_ref, b2_ref, w3_ref, b3_ref, g_ref, bg_ref,
                   out_ref):
    f32 = jnp.float32
    h = jnp.dot(nod_ref[...], wa_ref[...], preferred_element_type=f32)
    h += jnp.dot(acc_ref[...], wb_ref[...], preferred_element_type=f32)
    h = jnp.maximum(h + b1_ref[...], 0.0)
    h = jnp.maximum(
        jnp.dot(h, w2_ref[...], preferred_element_type=f32) + b2_ref[...], 0.0)
    h = jnp.dot(h, w3_ref[...], preferred_element_type=f32) + b3_ref[...]
    mu = jnp.mean(h, axis=-1, keepdims=True)
    d = h - mu
    var = jnp.mean(d * d, axis=-1, keepdims=True)
    out_ref[...] = d * lax.rsqrt(var + 1e-5) * g_ref[...] + bg_ref[...]


BN = 1000


@jax.jit
def _node_mlp(nodes, acc, wa, wb, b1, w2, b2, w3, b3, g, bg):
    row_spec = pl.BlockSpec((BN, H), lambda i: (i, 0))
    acc_spec = pl.BlockSpec((BN, H), lambda i: (i, 0))
    w_spec = pl.BlockSpec((H, H), lambda i: (0, 0))
    v_spec = pl.BlockSpec((1, H), lambda i: (0, 0))
    return pl.pallas_call(
        _node_mlp_body,
        grid=(N // BN,),
        in_specs=[row_spec, acc_spec,
                  w_spec, w_spec, v_spec,
                  w_spec, v_spec, w_spec, v_spec, v_spec, v_spec],
        out_specs=row_spec,
        out_shape=jax.ShapeDtypeStruct((N, H), jnp.float32),
    )(nodes, acc, wa, wb, b1, w2, b2, w3, b3, g, bg)


def kernel(nodes, edges, senders, receivers, We1, be1, We2, be2, We3, be3,
           ge, bge, Wn1, bn1, Wn2, bn2, Wn3, bn3, gn, bgn):
    senders = senders.astype(jnp.int32)
    receivers = receivers.astype(jnp.int32)
    r1 = lambda v: v.reshape(1, H)

    sr3d = jnp.stack([senders, receivers]).reshape(2, NW, NCH, CH)
    gath = _gather(nodes, sr3d)
    edges_out = _edge_mlp(
        gath, edges,
        We1[:H], We1[H:2 * H], We1[2 * H:], r1(be1),
        We2, r1(be2), We3, r1(be3), r1(ge), r1(bge))
    zeros = jnp.zeros((ACCR, H), jnp.float32)
    acc = _scatter(edges_out, receivers.reshape(NS, NCHT, CH), zeros)
    nodes_out = _node_mlp(
        nodes, acc,
        Wn1[:H], Wn1[H:], r1(bn1),
        Wn2, r1(bn2), Wn3, r1(bn3), r1(gn), r1(bgn))
    return (nodes_out, edges_out)


# speedup vs baseline: 2.8052x; 1.1405x over previous
import functools

import jax
import jax.numpy as jnp
from jax import lax
from jax.experimental import pallas as pl
from jax.experimental.pallas import tpu as pltpu
from jax.experimental.pallas import tpu_sc as plsc

N, E, H = 10000, 320000, 128
NC, NS = 2, 16
NW = NC * NS
K = 5
EK = E // K
NBUF = 5

GPW = EK // NW
CHG = 80
NCHG = GPW // CHG
CHS = 40
NCHS = GPW // CHS


@functools.cache
def _mesh():
    return plsc.VectorSubcoreMesh(core_axis_name="c", subcore_axis_name="s",
                                  num_cores=NC, num_subcores=NS)


def _worker_id():
    return lax.axis_index("c") * NS + lax.axis_index("s")


def _gather_body(nodes_hbm, sr_hbm, gout, idx_all, rows, gsems, wsems):
    wid = _worker_id()
    ebase = wid * GPW

    for k in range(2):
        pltpu.sync_copy(sr_hbm.at[k, wid], idx_all.at[k])

    def chunk_refs(j):
        sel = j // NCHG
        r = j % NCHG
        idx = idx_all.at[sel, r]
        out = gout.at[sel, pl.ds(ebase + r * CHG, CHG)]
        return idx, out

    for b in range(NBUF):
        idx, _ = chunk_refs(b)
        pltpu.async_copy(nodes_hbm.at[idx], rows[b], gsems[b])

    def group(jj, carry):
        for b in range(NBUF):
            j = jj * NBUF + b
            idx, out = chunk_refs(j)
            pltpu.make_async_copy(nodes_hbm.at[idx], rows[b], gsems[b]).wait()
            pltpu.async_copy(rows[b], out, wsems[b])
        for b in range(NBUF):
            j2 = (jj + 1) * NBUF + b
            @pl.when(j2 < 2 * NCHG)
            def _():
                idx2, out2 = chunk_refs(j2)
                pltpu.make_async_copy(rows[b], out2, wsems[b]).wait()
                pltpu.async_copy(nodes_hbm.at[idx2], rows[b], gsems[b])
        return carry

    lax.fori_loop(0, (2 * NCHG) // NBUF, group, 0)

    for b in range(NBUF):
        pltpu.make_async_copy(
            rows[b], gout.at[0, pl.ds(ebase, CHG)], wsems[b]).wait()


@jax.jit
def _gather(nodes, sr3d):
    return pl.kernel(
        _gather_body,
        out_type=jax.ShapeDtypeStruct((2, EK, H), jnp.float32),
        mesh=_mesh(),
        scratch_types=[
            pltpu.VMEM((2, NCHG, CHG), jnp.int32),
            [pltpu.VMEM((CHG, H), jnp.float32) for _ in range(NBUF)],
            [pltpu.SemaphoreType.DMA for _ in range(NBUF)],
            [pltpu.SemaphoreType.DMA for _ in range(NBUF)],
        ],
    )(nodes, sr3d)


def _scatter_body(eout_hbm, recv_hbm, zeros_hbm, acc_out,
                  idx_all, rows, lsems, eff_sh):
    wid = _worker_id()
    c = lax.axis_index("c")
    s = lax.axis_index("s")
    ebase = wid * GPW
    rpw = 624
    tail_off = rpw * NS
    tail = N - tail_off

    pltpu.sync_copy(recv_hbm.at[wid], idx_all)

    pltpu.sync_copy(zeros_hbm.at[pl.ds(s * rpw, rpw)],
                    eff_sh.at[pl.ds(s * rpw, rpw)])
    @pl.when(s == 0)
    def _():
        pltpu.sync_copy(zeros_hbm.at[pl.ds(tail_off, tail)],
                        eff_sh.at[pl.ds(tail_off, tail)])
    plsc.subcore_barrier()

    def load_ref(j):
        return eout_hbm.at[pl.ds(ebase + j * CHS, CHS)]

    for b in range(NBUF):
        pltpu.async_copy(load_ref(b), rows[b], lsems[b])

    def group(jj, carry):
        for b in range(NBUF):
            j = jj * NBUF + b
            pltpu.make_async_copy(load_ref(j), rows[b], lsems[b]).wait()
            pltpu.sync_copy(rows[b], eff_sh.at[idx_all.at[j]], add=True)
            j2 = j + NBUF
            @pl.when(j2 < NCHS)
            def _():
                pltpu.async_copy(load_ref(j2), rows[b], lsems[b])
        return carry

    lax.fori_loop(0, NCHS // NBUF, group, 0)
    plsc.subcore_barrier()

    pltpu.sync_copy(eff_sh.at[pl.ds(s * rpw, rpw)],
                    acc_out.at[c, pl.ds(s * rpw, rpw)])
    @pl.when(s == 0)
    def _():
        pltpu.sync_copy(eff_sh.at[pl.ds(tail_off, tail)],
                        acc_out.at[c, pl.ds(tail_off, tail)])


@jax.jit
def _scatter(eout_chunk, recv3d, zeros):
    return pl.kernel(
        _scatter_body,
        out_type=jax.ShapeDtypeStruct((NC, N, H), jnp.float32),
        mesh=_mesh(),
        scratch_types=[
            pltpu.VMEM((NCHS, CHS), jnp.int32),
            [pltpu.VMEM((CHS, H), jnp.float32) for _ in range(NBUF)],
            [pltpu.SemaphoreType.DMA for _ in range(NBUF)],
            pltpu.VMEM_SHARED((N, H), jnp.float32),
        ],
    )(eout_chunk, recv3d, zeros)


def _edge_mlp_body(big_ref, gath_src, gath_dst, edg_ref, wa_ref, wb_ref,
                   wc_ref, b1_ref, w2_ref, b2_ref, w3_ref, b3_ref, g_ref,
                   bg_ref, big_out, chunk_out):
    del big_ref
    f32 = jnp.float32
    h = jnp.dot(gath_dst[0], wa_ref[...], preferred_element_type=f32)
    h += jnp.dot(gath_src[0], wb_ref[...], preferred_element_type=f32)
    h += jnp.dot(edg_ref[...], wc_ref[...], preferred_element_type=f32)
    h = jnp.maximum(h + b1_ref[...], 0.0)
    h = jnp.maximum(
        jnp.dot(h, w2_ref[...], preferred_element_type=f32) + b2_ref[...], 0.0)
    h = jnp.dot(h, w3_ref[...], preferred_element_type=f32) + b3_ref[...]
    mu = jnp.mean(h, axis=-1, keepdims=True)
    d = h - mu
    var = jnp.mean(d * d, axis=-1, keepdims=True)
    out = d * lax.rsqrt(var + 1e-5) * g_ref[...] + bg_ref[...]
    big_out[...] = out
    chunk_out[...] = out


BE = 512
BPC = EK // BE


def _edge_mlp(k, big, gath, edg, wa, wb, wc, b1, w2, b2, w3, b3, g, bg):
    src_spec = pl.BlockSpec((1, BE, H), lambda i: (0, i, 0))
    dst_spec = pl.BlockSpec((1, BE, H), lambda i: (1, i, 0))
    edg_spec = pl.BlockSpec((BE, H), lambda i, _k=k: (_k * BPC + i, 0))
    big_spec = pl.BlockSpec(memory_space=pltpu.MemorySpace.HBM)
    w_spec = pl.BlockSpec((H, H), lambda i: (0, 0))
    v_spec = pl.BlockSpec((1, H), lambda i: (0, 0))
    return pl.pallas_call(
        _edge_mlp_body,
        grid=(BPC,),
        in_specs=[big_spec, src_spec, dst_spec, edg_spec,
                  w_spec, w_spec, w_spec, v_spec,
                  w_spec, v_spec, w_spec, v_spec, v_spec, v_spec],
        out_specs=[pl.BlockSpec((BE, H), lambda i, _k=k: (_k * BPC + i, 0)),
                   pl.BlockSpec((BE, H), lambda i: (i, 0))],
        out_shape=[jax.ShapeDtypeStruct((E, H), jnp.float32),
                   jax.ShapeDtypeStruct((EK, H), jnp.float32)],
        input_output_aliases={0: 0},
    )(big, gath, gath, edg, wa, wb, wc, b1, w2, b2, w3, b3, g, bg)


def _node_mlp_body(nod_ref, a0, a1, a2, a3, a4, wa_ref, wb_ref,
                   b1_ref, w2_ref, b2_ref, w3_ref, b3_ref, g_ref, bg_ref,
                   out_ref):
    f32 = jnp.float32
    eff = a0[0] + a0[1]
    for a in (a1, a2, a3, a4):
        eff += a[0] + a[1]
    h = jnp.dot(nod_ref[...], wa_ref[...], preferred_element_type=f32)
    h += jnp.dot(eff, wb_ref[...], preferred_element_type=f32)
    h = jnp.maximum(h + b1_ref[...], 0.0)
    h = jnp.maximum(
        jnp.dot(h, w2_ref[...], preferred_element_type=f32) + b2_ref[...], 0.0)
    h = jnp.dot(h, w3_ref[...], preferred_element_type=f32) + b3_ref[...]
    mu = jnp.mean(h, axis=-1, keepdims=True)
    d = h - mu
    var = jnp.mean(d * d, axis=-1, keepdims=True)
    out_ref[...] = d * lax.rsqrt(var + 1e-5) * g_ref[...] + bg_ref[...]


BN = 1000


def _node_mlp(nodes, accs, wa, wb, b1, w2, b2, w3, b3, g, bg):
    row_spec = pl.BlockSpec((BN, H), lambda i: (i, 0))
    acc_spec = pl.BlockSpec((NC, BN, H), lambda i: (0, i, 0))
    w_spec = pl.BlockSpec((H, H), lambda i: (0, 0))
    v_spec = pl.BlockSpec((1, H), lambda i: (0, 0))
    return pl.pallas_call(
        _node_mlp_body,
        grid=(N // BN,),
        in_specs=[row_spec] + [acc_spec] * K +
                 [w_spec, w_spec, v_spec,
                  w_spec, v_spec, w_spec, v_spec, v_spec, v_spec],
        out_specs=row_spec,
        out_shape=jax.ShapeDtypeStruct((N, H), jnp.float32),
    )(nodes, *accs, wa, wb, b1, w2, b2, w3, b3, g, bg)


def kernel(nodes, edges, senders, receivers, We1, be1, We2, be2, We3, be3,
           ge, bge, Wn1, bn1, Wn2, bn2, Wn3, bn3, gn, bgn):
    senders = senders.astype(jnp.int32)
    receivers = receivers.astype(jnp.int32)
    r1 = lambda v: v.reshape(1, H)

    sr = jnp.stack([senders, receivers]).reshape(2, K, NW, NCHG, CHG)
    recv4 = receivers.reshape(K, NW, NCHS, CHS)
    zeros = jnp.zeros((N, H), jnp.float32)
    big = jnp.zeros((E, H), jnp.float32)

    accs = []
    for k in range(K):
        gath = _gather(nodes, sr[:, k])
        big, eout_chunk = _edge_mlp(
            k, big, gath, edges,
            We1[:H], We1[H:2 * H], We1[2 * H:], r1(be1),
            We2, r1(be2), We3, r1(be3), r1(ge), r1(bge))
        accs.append(_scatter(eout_chunk, recv4[k], zeros))

    nodes_out = _node_mlp(
        nodes, accs,
        Wn1[:H], Wn1[H:], r1(bn1),
        Wn2, r1(bn2), Wn3, r1(bn3), r1(gn), r1(bgn))
    return (nodes_out, big)


# speedup vs baseline: 2.8269x; 1.0077x over previous
import functools

import jax
import jax.numpy as jnp
from jax import lax
from jax.experimental import pallas as pl
from jax.experimental.pallas import tpu as pltpu
from jax.experimental.pallas import tpu_sc as plsc

N, E, H = 10000, 320000, 128
NC, NS = 2, 16
NW = NC * NS
K = 5
EK = E // K
NBUF = 5

GPW = EK // NW
CHG = 80
NCHG = GPW // CHG
CHS = 40
NCHS = GPW // CHS


@functools.cache
def _mesh():
    return plsc.VectorSubcoreMesh(core_axis_name="c", subcore_axis_name="s",
                                  num_cores=NC, num_subcores=NS)


def _worker_id():
    return lax.axis_index("c") * NS + lax.axis_index("s")


def _gather_body(nodes_hbm, sr_hbm, gout, idx_all, rows, gsems, wsems):
    wid = _worker_id()
    ebase = wid * GPW

    for k in range(2):
        pltpu.sync_copy(sr_hbm.at[k, wid], idx_all.at[k])

    def chunk_refs(j):
        sel = j // NCHG
        r = j % NCHG
        idx = idx_all.at[sel, r]
        out = gout.at[sel, pl.ds(ebase + r * CHG, CHG)]
        return idx, out

    for b in range(NBUF):
        idx, _ = chunk_refs(b)
        pltpu.async_copy(nodes_hbm.at[idx], rows[b], gsems[b])

    def group(jj, carry):
        for b in range(NBUF):
            j = jj * NBUF + b
            idx, out = chunk_refs(j)
            pltpu.make_async_copy(nodes_hbm.at[idx], rows[b], gsems[b]).wait()
            pltpu.async_copy(rows[b], out, wsems[b])
        for b in range(NBUF):
            j2 = (jj + 1) * NBUF + b
            @pl.when(j2 < 2 * NCHG)
            def _():
                idx2, out2 = chunk_refs(j2)
                pltpu.make_async_copy(rows[b], out2, wsems[b]).wait()
                pltpu.async_copy(nodes_hbm.at[idx2], rows[b], gsems[b])
        return carry

    lax.fori_loop(0, (2 * NCHG) // NBUF, group, 0)

    for b in range(NBUF):
        pltpu.make_async_copy(
            rows[b], gout.at[0, pl.ds(ebase, CHG)], wsems[b]).wait()


@jax.jit
def _gather(nodes, sr3d):
    return pl.kernel(
        _gather_body,
        out_type=jax.ShapeDtypeStruct((2, EK, H), jnp.float32),
        mesh=_mesh(),
        scratch_types=[
            pltpu.VMEM((2, NCHG, CHG), jnp.int32),
            [pltpu.VMEM((CHG, H), jnp.float32) for _ in range(NBUF)],
            [pltpu.SemaphoreType.DMA for _ in range(NBUF)],
            [pltpu.SemaphoreType.DMA for _ in range(NBUF)],
        ],
    )(nodes, sr3d)


def _scatter_body(eout_hbm, recv_hbm, zeros_hbm, acc_out,
                  idx_all, rows, lsems, eff_sh):
    wid = _worker_id()
    c = lax.axis_index("c")
    s = lax.axis_index("s")
    ebase = wid * GPW
    rpw = 624
    tail_off = rpw * NS
    tail = N - tail_off

    pltpu.sync_copy(recv_hbm.at[wid], idx_all)

    pltpu.sync_copy(zeros_hbm.at[pl.ds(s * rpw, rpw)],
                    eff_sh.at[pl.ds(s * rpw, rpw)])
    @pl.when(s == 0)
    def _():
        pltpu.sync_copy(zeros_hbm.at[pl.ds(tail_off, tail)],
                        eff_sh.at[pl.ds(tail_off, tail)])
    plsc.subcore_barrier()

    def load_ref(j):
        return eout_hbm.at[pl.ds(ebase + j * CHS, CHS)]

    for b in range(NBUF):
        pltpu.async_copy(load_ref(b), rows[b], lsems[b])

    def group(jj, carry):
        for b in range(NBUF):
            j = jj * NBUF + b
            pltpu.make_async_copy(load_ref(j), rows[b], lsems[b]).wait()
            pltpu.sync_copy(rows[b], eff_sh.at[idx_all.at[j]], add=True)
            j2 = j + NBUF
            @pl.when(j2 < NCHS)
            def _():
                pltpu.async_copy(load_ref(j2), rows[b], lsems[b])
        return carry

    lax.fori_loop(0, NCHS // NBUF, group, 0)
    plsc.subcore_barrier()

    pltpu.sync_copy(eff_sh.at[pl.ds(s * rpw, rpw)],
                    acc_out.at[c, pl.ds(s * rpw, rpw)])
    @pl.when(s == 0)
    def _():
        pltpu.sync_copy(eff_sh.at[pl.ds(tail_off, tail)],
                        acc_out.at[c, pl.ds(tail_off, tail)])


@jax.jit
def _scatter(eout_chunk, recv3d, zeros):
    return pl.kernel(
        _scatter_body,
        out_type=jax.ShapeDtypeStruct((NC, N, H), jnp.float32),
        mesh=_mesh(),
        scratch_types=[
            pltpu.VMEM((NCHS, CHS), jnp.int32),
            [pltpu.VMEM((CHS, H), jnp.float32) for _ in range(NBUF)],
            [pltpu.SemaphoreType.DMA for _ in range(NBUF)],
            pltpu.VMEM_SHARED((N, H), jnp.float32),
        ],
    )(eout_chunk, recv3d, zeros)


def _edge_mlp_body(big_ref, gath_src, gath_dst, edg_ref, wa_ref, wb_ref,
                   wc_ref, b1_ref, w2_ref, b2_ref, w3_ref, b3_ref, g_ref,
                   bg_ref, big_out, chunk_out):
    del big_ref
    f32 = jnp.float32
    bf = lambda x: x.astype(jnp.bfloat16)
    h = jnp.dot(bf(gath_dst[0]), bf(wa_ref[...]), preferred_element_type=f32)
    h += jnp.dot(bf(gath_src[0]), bf(wb_ref[...]), preferred_element_type=f32)
    h += jnp.dot(bf(edg_ref[...]), bf(wc_ref[...]), preferred_element_type=f32)
    h = jnp.maximum(h + b1_ref[...], 0.0)
    h = jnp.maximum(
        jnp.dot(bf(h), bf(w2_ref[...]), preferred_element_type=f32)
        + b2_ref[...], 0.0)
    h = jnp.dot(bf(h), bf(w3_ref[...]), preferred_element_type=f32) + b3_ref[...]
    mu = jnp.mean(h, axis=-1, keepdims=True)
    d = h - mu
    var = jnp.mean(d * d, axis=-1, keepdims=True)
    out = d * lax.rsqrt(var + 1e-5) * g_ref[...] + bg_ref[...]
    big_out[...] = out
    chunk_out[...] = out


BE = 512
BPC = EK // BE


def _edge_mlp(k, big, gath, edg, wa, wb, wc, b1, w2, b2, w3, b3, g, bg):
    src_spec = pl.BlockSpec((1, BE, H), lambda i: (0, i, 0))
    dst_spec = pl.BlockSpec((1, BE, H), lambda i: (1, i, 0))
    edg_spec = pl.BlockSpec((BE, H), lambda i, _k=k: (_k * BPC + i, 0))
    big_spec = pl.BlockSpec(memory_space=pltpu.MemorySpace.HBM)
    w_spec = pl.BlockSpec((H, H), lambda i: (0, 0))
    v_spec = pl.BlockSpec((1, H), lambda i: (0, 0))
    return pl.pallas_call(
        _edge_mlp_body,
        grid=(BPC,),
        in_specs=[big_spec, src_spec, dst_spec, edg_spec,
                  w_spec, w_spec, w_spec, v_spec,
                  w_spec, v_spec, w_spec, v_spec, v_spec, v_spec],
        out_specs=[pl.BlockSpec((BE, H), lambda i, _k=k: (_k * BPC + i, 0)),
                   pl.BlockSpec((BE, H), lambda i: (i, 0))],
        out_shape=[jax.ShapeDtypeStruct((E, H), jnp.float32),
                   jax.ShapeDtypeStruct((EK, H), jnp.float32)],
        input_output_aliases={0: 0},
    )(big, gath, gath, edg, wa, wb, wc, b1, w2, b2, w3, b3, g, bg)


def _node_mlp_body(nod_ref, a0, a1, a2, a3, a4, wa_ref, wb_ref,
                   b1_ref, w2_ref, b2_ref, w3_ref, b3_ref, g_ref, bg_ref,
                   out_ref):
    f32 = jnp.float32
    eff = a0[0] + a0[1]
    for a in (a1, a2, a3, a4):
        eff += a[0] + a[1]
    h = jnp.dot(nod_ref[...], wa_ref[...], preferred_element_type=f32)
    h += jnp.dot(eff, wb_ref[...], preferred_element_type=f32)
    h = jnp.maximum(h + b1_ref[...], 0.0)
    h = jnp.maximum(
        jnp.dot(h, w2_ref[...], preferred_element_type=f32) + b2_ref[...], 0.0)
    h = jnp.dot(h, w3_ref[...], preferred_element_type=f32) + b3_ref[...]
    mu = jnp.mean(h, axis=-1, keepdims=True)
    d = h - mu
    var = jnp.mean(d * d, axis=-1, keepdims=True)
    out_ref[...] = d * lax.rsqrt(var + 1e-5) * g_ref[...] + bg_ref[...]


BN = 1000


def _node_mlp(nodes, accs, wa, wb, b1, w2, b2, w3, b3, g, bg):
    row_spec = pl.BlockSpec((BN, H), lambda i: (i, 0))
    acc_spec = pl.BlockSpec((NC, BN, H), lambda i: (0, i, 0))
    w_spec = pl.BlockSpec((H, H), lambda i: (0, 0))
    v_spec = pl.BlockSpec((1, H), lambda i: (0, 0))
    return pl.pallas_call(
        _node_mlp_body,
        grid=(N // BN,),
        in_specs=[row_spec] + [acc_spec] * K +
                 [w_spec, w_spec, v_spec,
                  w_spec, v_spec, w_spec, v_spec, v_spec, v_spec],
        out_specs=row_spec,
        out_shape=jax.ShapeDtypeStruct((N, H), jnp.float32),
    )(nodes, *accs, wa, wb, b1, w2, b2, w3, b3, g, bg)


def kernel(nodes, edges, senders, receivers, We1, be1, We2, be2, We3, be3,
           ge, bge, Wn1, bn1, Wn2, bn2, Wn3, bn3, gn, bgn):
    senders = senders.astype(jnp.int32)
    receivers = receivers.astype(jnp.int32)
    r1 = lambda v: v.reshape(1, H)

    sr = jnp.stack([senders, receivers]).reshape(2, K, NW, NCHG, CHG)
    recv4 = receivers.reshape(K, NW, NCHS, CHS)
    zeros = jnp.zeros((N, H), jnp.float32)
    big = jnp.zeros((E, H), jnp.float32)

    accs = []
    for k in range(K):
        gath = _gather(nodes, sr[:, k])
        big, eout_chunk = _edge_mlp(
            k, big, gath, edges,
            We1[:H], We1[H:2 * H], We1[2 * H:], r1(be1),
            We2, r1(be2), We3, r1(be3), r1(ge), r1(bge))
        accs.append(_scatter(eout_chunk, recv4[k], zeros))

    nodes_out = _node_mlp(
        nodes, accs,
        Wn1[:H], Wn1[H:], r1(bn1),
        Wn2, r1(bn2), Wn3, r1(bn3), r1(gn), r1(bgn))
    return (nodes_out, big)


# speedup vs baseline: 3.0745x; 1.0876x over previous
import functools

import jax
import jax.numpy as jnp
from jax import lax
from jax.experimental import pallas as pl
from jax.experimental.pallas import tpu as pltpu
from jax.experimental.pallas import tpu_sc as plsc

N, E, H = 10000, 320000, 128
NC, NS = 2, 16
NW = NC * NS
K = 5
EK = E // K
NBUF = 5

GPW = EK // NW
CHG = 80
NCHG = GPW // CHG
CHS = 40
NCHS = GPW // CHS


@functools.cache
def _mesh():
    return plsc.VectorSubcoreMesh(core_axis_name="c", subcore_axis_name="s",
                                  num_cores=NC, num_subcores=NS)


def _worker_id():
    return lax.axis_index("c") * NS + lax.axis_index("s")


def _gather_body(nodes_hbm, sr_hbm, gout, idx_all, rows, gsems, wsems):

    wid = _worker_id()
    ebase = wid * GPW

    for k in range(2):
        pltpu.sync_copy(sr_hbm.at[k, wid], idx_all.at[k])

    def chunk_refs(j):
        sel = j // NCHG
        r = j % NCHG
        idx = idx_all.at[sel, r]
        out = gout.at[sel, pl.ds(ebase + r * CHG, CHG)]
        return idx, out

    for b in range(NBUF):
        idx, _ = chunk_refs(b)
        pltpu.async_copy(nodes_hbm.at[idx], rows[b], gsems[b])

    def group(jj, carry):
        for b in range(NBUF):
            j = jj * NBUF + b
            idx, out = chunk_refs(j)
            pltpu.make_async_copy(nodes_hbm.at[idx], rows[b], gsems[b]).wait()
            pltpu.async_copy(rows[b], out, wsems[b])
        for b in range(NBUF):
            j2 = (jj + 1) * NBUF + b
            @pl.when(j2 < 2 * NCHG)
            def _():
                idx2, out2 = chunk_refs(j2)
                pltpu.make_async_copy(rows[b], out2, wsems[b]).wait()
                pltpu.async_copy(nodes_hbm.at[idx2], rows[b], gsems[b])
        return carry

    lax.fori_loop(0, (2 * NCHG) // NBUF, group, 0)

    for b in range(NBUF):
        pltpu.make_async_copy(
            rows[b], gout.at[0, pl.ds(ebase, CHG)], wsems[b]).wait()


@jax.jit
def _gather(nodes_pk, sr3d):
    return pl.kernel(
        _gather_body,
        out_type=jax.ShapeDtypeStruct((2, EK, H), jnp.float32),
        mesh=_mesh(),
        scratch_types=[
            pltpu.VMEM((2, NCHG, CHG), jnp.int32),
            [pltpu.VMEM((CHG, H), jnp.float32) for _ in range(NBUF)],
            [pltpu.SemaphoreType.DMA for _ in range(NBUF)],
            [pltpu.SemaphoreType.DMA for _ in range(NBUF)],
        ],
    )(nodes_pk, sr3d)


def _scatter_body(eout_hbm, recv_hbm, zeros_hbm, acc_out,
                  idx_all, rows, lsems, eff_sh):
    wid = _worker_id()
    c = lax.axis_index("c")
    s = lax.axis_index("s")
    ebase = wid * GPW
    rpw = 624
    tail_off = rpw * NS
    tail = N - tail_off

    pltpu.sync_copy(recv_hbm.at[wid], idx_all)

    pltpu.sync_copy(zeros_hbm.at[pl.ds(s * rpw, rpw)],
                    eff_sh.at[pl.ds(s * rpw, rpw)])
    @pl.when(s == 0)
    def _():
        pltpu.sync_copy(zeros_hbm.at[pl.ds(tail_off, tail)],
                        eff_sh.at[pl.ds(tail_off, tail)])
    plsc.subcore_barrier()

    def load_ref(j):
        return eout_hbm.at[pl.ds(ebase + j * CHS, CHS)]

    for b in range(NBUF):
        pltpu.async_copy(load_ref(b), rows[b], lsems[b])

    def group(jj, carry):
        for b in range(NBUF):
            j = jj * NBUF + b
            pltpu.make_async_copy(load_ref(j), rows[b], lsems[b]).wait()
            pltpu.sync_copy(rows[b], eff_sh.at[idx_all.at[j]], add=True)
            j2 = j + NBUF
            @pl.when(j2 < NCHS)
            def _():
                pltpu.async_copy(load_ref(j2), rows[b], lsems[b])
        return carry

    lax.fori_loop(0, NCHS // NBUF, group, 0)
    plsc.subcore_barrier()

    pltpu.sync_copy(eff_sh.at[pl.ds(s * rpw, rpw)],
                    acc_out.at[c, pl.ds(s * rpw, rpw)])
    @pl.when(s == 0)
    def _():
        pltpu.sync_copy(eff_sh.at[pl.ds(tail_off, tail)],
                        acc_out.at[c, pl.ds(tail_off, tail)])


@jax.jit
def _scatter(eout_chunk, recv3d, zeros):
    return pl.kernel(
        _scatter_body,
        out_type=jax.ShapeDtypeStruct((NC, N, H), jnp.float32),
        mesh=_mesh(),
        scratch_types=[
            pltpu.VMEM((NCHS, CHS), jnp.int32),
            [pltpu.VMEM((CHS, H), jnp.float32) for _ in range(NBUF)],
            [pltpu.SemaphoreType.DMA for _ in range(NBUF)],
            pltpu.VMEM_SHARED((N, H), jnp.float32),
        ],
    )(eout_chunk, recv3d, zeros)


def _edge_mlp_body(gath_src, gath_dst, edg_ref, wa_ref, wb_ref,
                   wc_ref, b1_ref, w2_ref, b2_ref, w3_ref, b3_ref, g_ref,
                   bg_ref, big_out, chunk_out):
    f32 = jnp.float32
    bf = lambda x: x.astype(jnp.bfloat16)
    h = jnp.dot(bf(gath_dst[0]), bf(wa_ref[...]), preferred_element_type=f32)
    h += jnp.dot(bf(gath_src[0]), bf(wb_ref[...]), preferred_element_type=f32)
    h += jnp.dot(bf(edg_ref[...]), bf(wc_ref[...]), preferred_element_type=f32)
    h = jnp.maximum(h + b1_ref[...], 0.0)
    h = jnp.maximum(
        jnp.dot(bf(h), bf(w2_ref[...]), preferred_element_type=f32)
        + b2_ref[...], 0.0)
    h = jnp.dot(bf(h), bf(w3_ref[...]), preferred_element_type=f32) + b3_ref[...]
    mu = jnp.mean(h, axis=-1, keepdims=True)
    d = h - mu
    var = jnp.mean(d * d, axis=-1, keepdims=True)
    out = d * lax.rsqrt(var + 1e-5) * g_ref[...] + bg_ref[...]
    big_out[...] = out
    chunk_out[...] = out


BE = 512
BPC = EK // BE


def _edge_mlp(k, big, gath, edg, wa, wb, wc, b1, w2, b2, w3, b3, g, bg):
    src_spec = pl.BlockSpec((1, BE, H), lambda i: (0, i, 0))
    dst_spec = pl.BlockSpec((1, BE, H), lambda i: (1, i, 0))
    edg_spec = pl.BlockSpec((BE, H), lambda i, _k=k: (_k * BPC + i, 0))
    big_spec = pl.BlockSpec(memory_space=pltpu.MemorySpace.HBM)
    w_spec = pl.BlockSpec((H, H), lambda i: (0, 0))
    v_spec = pl.BlockSpec((1, H), lambda i: (0, 0))
    body = _edge_mlp_body
    in_specs = [src_spec, dst_spec, edg_spec,
                w_spec, w_spec, w_spec, v_spec,
                w_spec, v_spec, w_spec, v_spec, v_spec, v_spec]
    args = (gath, gath, edg, wa, wb, wc, b1, w2, b2, w3, b3, g, bg)
    aliases = {}
    if k > 0:
        body = lambda big_ref, *rest: _edge_mlp_body(*rest)
        in_specs = [big_spec] + in_specs
        args = (big,) + args
        aliases = {0: 0}
    return pl.pallas_call(
        body,
        grid=(BPC,),
        in_specs=in_specs,
        out_specs=[pl.BlockSpec((BE, H), lambda i, _k=k: (_k * BPC + i, 0)),
                   pl.BlockSpec((BE, H), lambda i: (i, 0))],
        out_shape=[jax.ShapeDtypeStruct((E, H), jnp.float32),
                   jax.ShapeDtypeStruct((EK, H), jnp.float32)],
        input_output_aliases=aliases,
    )(*args)


def _node_mlp_body(nod_ref, a0, a1, a2, a3, a4, wa_ref, wb_ref,
                   b1_ref, w2_ref, b2_ref, w3_ref, b3_ref, g_ref, bg_ref,
                   out_ref):
    f32 = jnp.float32
    eff = a0[0] + a0[1]
    for a in (a1, a2, a3, a4):
        eff += a[0] + a[1]
    h = jnp.dot(nod_ref[...], wa_ref[...], preferred_element_type=f32)
    h += jnp.dot(eff, wb_ref[...], preferred_element_type=f32)
    h = jnp.maximum(h + b1_ref[...], 0.0)
    h = jnp.maximum(
        jnp.dot(h, w2_ref[...], preferred_element_type=f32) + b2_ref[...], 0.0)
    h = jnp.dot(h, w3_ref[...], preferred_element_type=f32) + b3_ref[...]
    mu = jnp.mean(h, axis=-1, keepdims=True)
    d = h - mu
    var = jnp.mean(d * d, axis=-1, keepdims=True)
    out_ref[...] = d * lax.rsqrt(var + 1e-5) * g_ref[...] + bg_ref[...]


BN = 1000


def _node_mlp(nodes, accs, wa, wb, b1, w2, b2, w3, b3, g, bg):
    row_spec = pl.BlockSpec((BN, H), lambda i: (i, 0))
    acc_spec = pl.BlockSpec((NC, BN, H), lambda i: (0, i, 0))
    w_spec = pl.BlockSpec((H, H), lambda i: (0, 0))
    v_spec = pl.BlockSpec((1, H), lambda i: (0, 0))
    return pl.pallas_call(
        _node_mlp_body,
        grid=(N // BN,),
        in_specs=[row_spec] + [acc_spec] * K +
                 [w_spec, w_spec, v_spec,
                  w_spec, v_spec, w_spec, v_spec, v_spec, v_spec],
        out_specs=row_spec,
        out_shape=jax.ShapeDtypeStruct((N, H), jnp.float32),
    )(nodes, *accs, wa, wb, b1, w2, b2, w3, b3, g, bg)


def kernel(nodes, edges, senders, receivers, We1, be1, We2, be2, We3, be3,
           ge, bge, Wn1, bn1, Wn2, bn2, Wn3, bn3, gn, bgn):
    senders = senders.astype(jnp.int32)
    receivers = receivers.astype(jnp.int32)
    r1 = lambda v: v.reshape(1, H)

    sr = jnp.stack([senders, receivers]).reshape(2, K, NW, NCHG, CHG)
    recv4 = receivers.reshape(K, NW, NCHS, CHS)
    zeros = jnp.zeros((N, H), jnp.float32)
    big = None

    accs = []
    for k in range(K):
        gath = _gather(nodes, sr[:, k])
        big, eout_chunk = _edge_mlp(
            k, big, gath, edges,
            We1[:H], We1[H:2 * H], We1[2 * H:], r1(be1),
            We2, r1(be2), We3, r1(be3), r1(ge), r1(bge))
        accs.append(_scatter(eout_chunk, recv4[k], zeros))

    nodes_out = _node_mlp(
        nodes, accs,
        Wn1[:H], Wn1[H:], r1(bn1),
        Wn2, r1(bn2), Wn3, r1(bn3), r1(gn), r1(bgn))
    return (nodes_out, big)


# speedup vs baseline: 3.7078x; 1.2060x over previous
import functools

import jax
import jax.numpy as jnp
from jax import lax
from jax.experimental import pallas as pl
from jax.experimental.pallas import tpu as pltpu
from jax.experimental.pallas import tpu_sc as plsc

N, E, H = 10000, 320000, 128
NC, NS = 2, 16
NW = NC * NS
K = 5
EK = E // K
NBUF = 5

GPW = EK // NW
CHG = 80
NCHG = GPW // CHG
CHS = 40
NCHS = GPW // CHS


@functools.cache
def _mesh():
    return plsc.VectorSubcoreMesh(core_axis_name="c", subcore_axis_name="s",
                                  num_cores=NC, num_subcores=NS)


def _worker_id():
    return lax.axis_index("c") * NS + lax.axis_index("s")


def _gather_body(nodes_hbm, sr_hbm, tok_hbm, gout, idx_all, rows, gsems, wsems):
    del tok_hbm

    wid = _worker_id()
    ebase = wid * GPW

    for k in range(2):
        pltpu.sync_copy(sr_hbm.at[k, wid], idx_all.at[k])

    def chunk_refs(j):
        sel = j // NCHG
        r = j % NCHG
        idx = idx_all.at[sel, r]
        out = gout.at[sel, pl.ds(ebase + r * CHG, CHG)]
        return idx, out

    for b in range(NBUF):
        idx, _ = chunk_refs(b)
        pltpu.async_copy(nodes_hbm.at[idx], rows[b], gsems[b])

    def group(jj, carry):
        for b in range(NBUF):
            j = jj * NBUF + b
            idx, out = chunk_refs(j)
            pltpu.make_async_copy(nodes_hbm.at[idx], rows[b], gsems[b]).wait()
            pltpu.async_copy(rows[b], out, wsems[b])
        for b in range(NBUF):
            j2 = (jj + 1) * NBUF + b
            @pl.when(j2 < 2 * NCHG)
            def _():
                idx2, out2 = chunk_refs(j2)
                pltpu.make_async_copy(rows[b], out2, wsems[b]).wait()
                pltpu.async_copy(nodes_hbm.at[idx2], rows[b], gsems[b])
        return carry

    lax.fori_loop(0, (2 * NCHG) // NBUF, group, 0)

    for b in range(NBUF):
        pltpu.make_async_copy(
            rows[b], gout.at[0, pl.ds(ebase, CHG)], wsems[b]).wait()


@jax.jit
def _gather(nodes, sr3d, tok):
    return pl.kernel(
        _gather_body,
        out_type=jax.ShapeDtypeStruct((2, EK, H), jnp.float32),
        mesh=_mesh(),
        scratch_types=[
            pltpu.VMEM((2, NCHG, CHG), jnp.int32),
            [pltpu.VMEM((CHG, H), jnp.float32) for _ in range(NBUF)],
            [pltpu.SemaphoreType.DMA for _ in range(NBUF)],
            [pltpu.SemaphoreType.DMA for _ in range(NBUF)],
        ],
    )(nodes, sr3d, tok)


def _scatter_body(eout_hbm, recv_hbm, zeros_hbm, acc_out,
                  idx_all, rows, lsems, eff_sh):
    wid = _worker_id()
    c = lax.axis_index("c")
    s = lax.axis_index("s")
    ebase = wid * GPW
    rpw = 624
    tail_off = rpw * NS
    tail = N - tail_off

    pltpu.sync_copy(recv_hbm.at[wid], idx_all)

    pltpu.sync_copy(zeros_hbm.at[pl.ds(s * rpw, rpw)],
                    eff_sh.at[pl.ds(s * rpw, rpw)])
    @pl.when(s == 0)
    def _():
        pltpu.sync_copy(zeros_hbm.at[pl.ds(tail_off, tail)],
                        eff_sh.at[pl.ds(tail_off, tail)])
    plsc.subcore_barrier()

    def load_ref(j):
        return eout_hbm.at[pl.ds(ebase + j * CHS, CHS)]

    for b in range(NBUF):
        pltpu.async_copy(load_ref(b), rows[b], lsems[b])

    def group(jj, carry):
        for b in range(NBUF):
            j = jj * NBUF + b
            pltpu.make_async_copy(load_ref(j), rows[b], lsems[b]).wait()
            pltpu.sync_copy(rows[b], eff_sh.at[idx_all.at[j]], add=True)
            j2 = j + NBUF
            @pl.when(j2 < NCHS)
            def _():
                pltpu.async_copy(load_ref(j2), rows[b], lsems[b])
        return carry

    lax.fori_loop(0, NCHS // NBUF, group, 0)
    plsc.subcore_barrier()

    pltpu.sync_copy(eff_sh.at[pl.ds(s * rpw, rpw)],
                    acc_out.at[c, pl.ds(s * rpw, rpw)])
    @pl.when(s == 0)
    def _():
        pltpu.sync_copy(eff_sh.at[pl.ds(tail_off, tail)],
                        acc_out.at[c, pl.ds(tail_off, tail)])


@jax.jit
def _scatter(eout_chunk, recv3d, zeros):
    return pl.kernel(
        _scatter_body,
        out_type=jax.ShapeDtypeStruct((NC, N, H), jnp.float32),
        mesh=_mesh(),
        scratch_types=[
            pltpu.VMEM((NCHS, CHS), jnp.int32),
            [pltpu.VMEM((CHS, H), jnp.float32) for _ in range(NBUF)],
            [pltpu.SemaphoreType.DMA for _ in range(NBUF)],
            pltpu.VMEM_SHARED((N, H), jnp.float32),
        ],
    )(eout_chunk, recv3d, zeros)


def _edge_mlp_body(gath_src, gath_dst, edg_ref, wa_ref, wb_ref,
                   wc_ref, b1_ref, w2_ref, b2_ref, w3_ref, b3_ref, g_ref,
                   bg_ref, big_out, chunk_out):
    f32 = jnp.float32
    bf = lambda x: x.astype(jnp.bfloat16)
    h = jnp.dot(bf(gath_dst[0]), bf(wa_ref[...]), preferred_element_type=f32)
    h += jnp.dot(bf(gath_src[0]), bf(wb_ref[...]), preferred_element_type=f32)
    h += jnp.dot(bf(edg_ref[...]), bf(wc_ref[...]), preferred_element_type=f32)
    h = jnp.maximum(h + b1_ref[...], 0.0)
    h = jnp.maximum(
        jnp.dot(bf(h), bf(w2_ref[...]), preferred_element_type=f32)
        + b2_ref[...], 0.0)
    h = jnp.dot(bf(h), bf(w3_ref[...]), preferred_element_type=f32) + b3_ref[...]
    mu = jnp.mean(h, axis=-1, keepdims=True)
    d = h - mu
    var = jnp.mean(d * d, axis=-1, keepdims=True)
    out = d * lax.rsqrt(var + 1e-5) * g_ref[...] + bg_ref[...]
    big_out[...] = out
    chunk_out[...] = out


BE = 800
BPC = EK // BE


def _edge_mlp(k, big, gath, edg, wa, wb, wc, b1, w2, b2, w3, b3, g, bg):
    src_spec = pl.BlockSpec((1, BE, H), lambda i: (0, i, 0))
    dst_spec = pl.BlockSpec((1, BE, H), lambda i: (1, i, 0))
    edg_spec = pl.BlockSpec((BE, H), lambda i, _k=k: (_k * BPC + i, 0))
    big_spec = pl.BlockSpec(memory_space=pltpu.MemorySpace.HBM)
    w_spec = pl.BlockSpec((H, H), lambda i: (0, 0))
    v_spec = pl.BlockSpec((1, H), lambda i: (0, 0))
    body = _edge_mlp_body
    in_specs = [src_spec, dst_spec, edg_spec,
                w_spec, w_spec, w_spec, v_spec,
                w_spec, v_spec, w_spec, v_spec, v_spec, v_spec]
    args = (gath, gath, edg, wa, wb, wc, b1, w2, b2, w3, b3, g, bg)
    aliases = {}
    if k > 0:
        body = lambda big_ref, *rest: _edge_mlp_body(*rest)
        in_specs = [big_spec] + in_specs
        args = (big,) + args
        aliases = {0: 0}
    return pl.pallas_call(
        body,
        grid=(BPC,),
        in_specs=in_specs,
        out_specs=[pl.BlockSpec((BE, H), lambda i, _k=k: (_k * BPC + i, 0)),
                   pl.BlockSpec((BE, H), lambda i: (i, 0))],
        out_shape=[jax.ShapeDtypeStruct((E, H), jnp.float32),
                   jax.ShapeDtypeStruct((EK, H), jnp.float32)],
        input_output_aliases=aliases,
    )(*args)


def _node_mlp_body(nod_ref, a0, a1, a2, a3, a4, wa_ref, wb_ref,
                   b1_ref, w2_ref, b2_ref, w3_ref, b3_ref, g_ref, bg_ref,
                   out_ref):
    f32 = jnp.float32
    eff = a0[0] + a0[1]
    for a in (a1, a2, a3, a4):
        eff += a[0] + a[1]
    h = jnp.dot(nod_ref[...], wa_ref[...], preferred_element_type=f32)
    h += jnp.dot(eff, wb_ref[...], preferred_element_type=f32)
    h = jnp.maximum(h + b1_ref[...], 0.0)
    h = jnp.maximum(
        jnp.dot(h, w2_ref[...], preferred_element_type=f32) + b2_ref[...], 0.0)
    h = jnp.dot(h, w3_ref[...], preferred_element_type=f32) + b3_ref[...]
    mu = jnp.mean(h, axis=-1, keepdims=True)
    d = h - mu
    var = jnp.mean(d * d, axis=-1, keepdims=True)
    out_ref[...] = d * lax.rsqrt(var + 1e-5) * g_ref[...] + bg_ref[...]


BN = 1000


def _node_mlp(nodes, accs, wa, wb, b1, w2, b2, w3, b3, g, bg):
    row_spec = pl.BlockSpec((BN, H), lambda i: (i, 0))
    acc_spec = pl.BlockSpec((NC, BN, H), lambda i: (0, i, 0))
    w_spec = pl.BlockSpec((H, H), lambda i: (0, 0))
    v_spec = pl.BlockSpec((1, H), lambda i: (0, 0))
    return pl.pallas_call(
        _node_mlp_body,
        grid=(N // BN,),
        in_specs=[row_spec] + [acc_spec] * K +
                 [w_spec, w_spec, v_spec,
                  w_spec, v_spec, w_spec, v_spec, v_spec, v_spec],
        out_specs=row_spec,
        out_shape=jax.ShapeDtypeStruct((N, H), jnp.float32),
    )(nodes, *accs, wa, wb, b1, w2, b2, w3, b3, g, bg)


def kernel(nodes, edges, senders, receivers, We1, be1, We2, be2, We3, be3,
           ge, bge, Wn1, bn1, Wn2, bn2, Wn3, bn3, gn, bgn):
    senders = senders.astype(jnp.int32)
    receivers = receivers.astype(jnp.int32)
    r1 = lambda v: v.reshape(1, H)

    sr = jnp.stack([senders, receivers]).reshape(2, K, NW, NCHG, CHG)
    recv4 = receivers.reshape(K, NW, NCHS, CHS)
    zeros = jnp.zeros((N, H), jnp.float32)
    big = None

    accs = []
    eouts = []
    for k in range(K):
        tok = eouts[k - 2] if k >= 2 else senders
        gath = _gather(nodes, sr[:, k], tok)
        big, eout_chunk = _edge_mlp(
            k, big, gath, edges,
            We1[:H], We1[H:2 * H], We1[2 * H:], r1(be1),
            We2, r1(be2), We3, r1(be3), r1(ge), r1(bge))
        eouts.append(eout_chunk)
        accs.append(_scatter(eout_chunk, recv4[k], zeros))

    nodes_out = _node_mlp(
        nodes, accs,
        Wn1[:H], Wn1[H:], r1(bn1),
        Wn2, r1(bn2), Wn3, r1(bn3), r1(gn), r1(bgn))
    return (nodes_out, big)


# speedup vs baseline: 4.1941x; 1.1312x over previous
import functools

import jax
import jax.numpy as jnp
from jax import lax
from jax.experimental import pallas as pl
from jax.experimental.pallas import tpu as pltpu
from jax.experimental.pallas import tpu_sc as plsc

N, E, H = 10000, 320000, 128
NC, NS = 2, 16
NW = NC * NS
K = 5
EK = E // K
NBUF = 5

GPW = EK // NW
CHG = 80
NCHG = GPW // CHG
CHS = 40
NCHS = GPW // CHS


@functools.cache
def _mesh():
    return plsc.VectorSubcoreMesh(core_axis_name="c", subcore_axis_name="s",
                                  num_cores=NC, num_subcores=NS)


def _worker_id():
    return lax.axis_index("c") * NS + lax.axis_index("s")


def _gather_body(nodes_hbm, sr_hbm, tok_hbm, gout, idx_all, rows, gsems, wsems):
    del tok_hbm

    wid = _worker_id()
    ebase = wid * GPW

    for k in range(2):
        pltpu.sync_copy(sr_hbm.at[k, wid], idx_all.at[k])

    def chunk_refs(j):
        sel = j // NCHG
        r = j % NCHG
        idx = idx_all.at[sel, r]
        out = gout.at[sel, pl.ds(ebase + r * CHG, CHG)]
        return idx, out

    for b in range(NBUF):
        idx, _ = chunk_refs(b)
        pltpu.async_copy(nodes_hbm.at[idx], rows[b], gsems[b])

    def group(jj, carry):
        for b in range(NBUF):
            j = jj * NBUF + b
            idx, out = chunk_refs(j)
            pltpu.make_async_copy(nodes_hbm.at[idx], rows[b], gsems[b]).wait()
            pltpu.async_copy(rows[b], out, wsems[b])
        for b in range(NBUF):
            j2 = (jj + 1) * NBUF + b
            @pl.when(j2 < 2 * NCHG)
            def _():
                idx2, out2 = chunk_refs(j2)
                pltpu.make_async_copy(rows[b], out2, wsems[b]).wait()
                pltpu.async_copy(nodes_hbm.at[idx2], rows[b], gsems[b])
        return carry

    lax.fori_loop(0, (2 * NCHG) // NBUF, group, 0)

    for b in range(NBUF):
        pltpu.make_async_copy(
            rows[b], gout.at[0, pl.ds(ebase, CHG)], wsems[b]).wait()


@jax.jit
def _gather(nodes, sr3d, tok):
    return pl.kernel(
        _gather_body,
        out_type=jax.ShapeDtypeStruct((2, EK, H), jnp.float32),
        mesh=_mesh(),
        scratch_types=[
            pltpu.VMEM((2, NCHG, CHG), jnp.int32),
            [pltpu.VMEM((CHG, H), jnp.float32) for _ in range(NBUF)],
            [pltpu.SemaphoreType.DMA for _ in range(NBUF)],
            [pltpu.SemaphoreType.DMA for _ in range(NBUF)],
        ],
    )(nodes, sr3d, tok)


def _scatter_body(eout_hbm, recv_hbm, zeros_hbm, acc_out,
                  idx_all, rows, lsems, eff_sh):
    wid = _worker_id()
    c = lax.axis_index("c")
    s = lax.axis_index("s")
    ebase = wid * GPW
    rpw = 624
    tail_off = rpw * NS
    tail = N - tail_off

    pltpu.sync_copy(recv_hbm.at[wid], idx_all)

    pltpu.sync_copy(zeros_hbm.at[pl.ds(s * rpw, rpw)],
                    eff_sh.at[pl.ds(s * rpw, rpw)])
    @pl.when(s == 0)
    def _():
        pltpu.sync_copy(zeros_hbm.at[pl.ds(tail_off, tail)],
                        eff_sh.at[pl.ds(tail_off, tail)])
    plsc.subcore_barrier()

    def load_ref(j):
        return eout_hbm.at[pl.ds(ebase + j * CHS, CHS)]

    for b in range(NBUF):
        pltpu.async_copy(load_ref(b), rows[b], lsems[b])

    def group(jj, carry):
        for b in range(NBUF):
            j = jj * NBUF + b
            pltpu.make_async_copy(load_ref(j), rows[b], lsems[b]).wait()
            pltpu.sync_copy(rows[b], eff_sh.at[idx_all.at[j]], add=True)
            j2 = j + NBUF
            @pl.when(j2 < NCHS)
            def _():
                pltpu.async_copy(load_ref(j2), rows[b], lsems[b])
        return carry

    lax.fori_loop(0, NCHS // NBUF, group, 0)
    plsc.subcore_barrier()

    pltpu.sync_copy(eff_sh.at[pl.ds(s * rpw, rpw)],
                    acc_out.at[c, pl.ds(s * rpw, rpw)])
    @pl.when(s == 0)
    def _():
        pltpu.sync_copy(eff_sh.at[pl.ds(tail_off, tail)],
                        acc_out.at[c, pl.ds(tail_off, tail)])


@jax.jit
def _scatter(eout_chunk, recv3d, zeros):
    return pl.kernel(
        _scatter_body,
        out_type=jax.ShapeDtypeStruct((NC, N, H), jnp.float32),
        mesh=_mesh(),
        scratch_types=[
            pltpu.VMEM((NCHS, CHS), jnp.int32),
            [pltpu.VMEM((CHS, H), jnp.float32) for _ in range(NBUF)],
            [pltpu.SemaphoreType.DMA for _ in range(NBUF)],
            pltpu.VMEM_SHARED((N, H), jnp.float32),
        ],
    )(eout_chunk, recv3d, zeros)


def _edge_mlp_body(gath_src, gath_dst, edg_ref, wa_ref, wb_ref,
                   wc_ref, b1_ref, w2_ref, b2_ref, w3_ref, b3_ref, g_ref,
                   bg_ref, big_out, chunk_out):
    f32 = jnp.float32
    bf = lambda x: x.astype(jnp.bfloat16)
    h = jnp.dot(bf(gath_dst[0]), bf(wa_ref[...]), preferred_element_type=f32)
    h += jnp.dot(bf(gath_src[0]), bf(wb_ref[...]), preferred_element_type=f32)
    h += jnp.dot(bf(edg_ref[...]), bf(wc_ref[...]), preferred_element_type=f32)
    h = jnp.maximum(h + b1_ref[...], 0.0)
    h = jnp.maximum(
        jnp.dot(bf(h), bf(w2_ref[...]), preferred_element_type=f32)
        + b2_ref[...], 0.0)
    h = jnp.dot(bf(h), bf(w3_ref[...]), preferred_element_type=f32) + b3_ref[...]
    mu = jnp.mean(h, axis=-1, keepdims=True)
    d = h - mu
    var = jnp.mean(d * d, axis=-1, keepdims=True)
    out = d * lax.rsqrt(var + 1e-5) * g_ref[...] + bg_ref[...]
    big_out[...] = out
    chunk_out[...] = out


BE = 1600
BPC = EK // BE


def _edge_mlp(k, big, gath, edg, wa, wb, wc, b1, w2, b2, w3, b3, g, bg):
    src_spec = pl.BlockSpec((1, BE, H), lambda i: (0, i, 0))
    dst_spec = pl.BlockSpec((1, BE, H), lambda i: (1, i, 0))
    edg_spec = pl.BlockSpec((BE, H), lambda i, _k=k: (_k * BPC + i, 0))
    big_spec = pl.BlockSpec(memory_space=pltpu.MemorySpace.HBM)
    w_spec = pl.BlockSpec((H, H), lambda i: (0, 0))
    v_spec = pl.BlockSpec((1, H), lambda i: (0, 0))
    body = _edge_mlp_body
    in_specs = [src_spec, dst_spec, edg_spec,
                w_spec, w_spec, w_spec, v_spec,
                w_spec, v_spec, w_spec, v_spec, v_spec, v_spec]
    args = (gath, gath, edg, wa, wb, wc, b1, w2, b2, w3, b3, g, bg)
    aliases = {}
    if k > 0:
        body = lambda big_ref, *rest: _edge_mlp_body(*rest)
        in_specs = [big_spec] + in_specs
        args = (big,) + args
        aliases = {0: 0}
    return pl.pallas_call(
        body,
        grid=(BPC,),
        in_specs=in_specs,
        out_specs=[pl.BlockSpec((BE, H), lambda i, _k=k: (_k * BPC + i, 0)),
                   pl.BlockSpec((BE, H), lambda i: (i, 0))],
        out_shape=[jax.ShapeDtypeStruct((E, H), jnp.float32),
                   jax.ShapeDtypeStruct((EK, H), jnp.float32)],
        input_output_aliases=aliases,
    )(*args)


def _node_mlp_body(nod_ref, a0, a1, a2, a3, a4, wa_ref, wb_ref,
                   b1_ref, w2_ref, b2_ref, w3_ref, b3_ref, g_ref, bg_ref,
                   out_ref):
    f32 = jnp.float32
    eff = a0[0] + a0[1]
    for a in (a1, a2, a3, a4):
        eff += a[0] + a[1]
    h = jnp.dot(nod_ref[...], wa_ref[...], preferred_element_type=f32)
    h += jnp.dot(eff, wb_ref[...], preferred_element_type=f32)
    h = jnp.maximum(h + b1_ref[...], 0.0)
    h = jnp.maximum(
        jnp.dot(h, w2_ref[...], preferred_element_type=f32) + b2_ref[...], 0.0)
    h = jnp.dot(h, w3_ref[...], preferred_element_type=f32) + b3_ref[...]
    mu = jnp.mean(h, axis=-1, keepdims=True)
    d = h - mu
    var = jnp.mean(d * d, axis=-1, keepdims=True)
    out_ref[...] = d * lax.rsqrt(var + 1e-5) * g_ref[...] + bg_ref[...]


BN = 1000


def _node_mlp(nodes, accs, wa, wb, b1, w2, b2, w3, b3, g, bg):
    row_spec = pl.BlockSpec((BN, H), lambda i: (i, 0))
    acc_spec = pl.BlockSpec((NC, BN, H), lambda i: (0, i, 0))
    w_spec = pl.BlockSpec((H, H), lambda i: (0, 0))
    v_spec = pl.BlockSpec((1, H), lambda i: (0, 0))
    return pl.pallas_call(
        _node_mlp_body,
        grid=(N // BN,),
        in_specs=[row_spec] + [acc_spec] * K +
                 [w_spec, w_spec, v_spec,
                  w_spec, v_spec, w_spec, v_spec, v_spec, v_spec],
        out_specs=row_spec,
        out_shape=jax.ShapeDtypeStruct((N, H), jnp.float32),
    )(nodes, *accs, wa, wb, b1, w2, b2, w3, b3, g, bg)


def kernel(nodes, edges, senders, receivers, We1, be1, We2, be2, We3, be3,
           ge, bge, Wn1, bn1, Wn2, bn2, Wn3, bn3, gn, bgn):
    senders = senders.astype(jnp.int32)
    receivers = receivers.astype(jnp.int32)
    r1 = lambda v: v.reshape(1, H)

    sr = jnp.stack([senders, receivers]).reshape(2, K, NW, NCHG, CHG)
    recv4 = receivers.reshape(K, NW, NCHS, CHS)
    zeros = jnp.zeros((N, H), jnp.float32)
    big = None

    accs = []
    eouts = []
    for k in range(K):
        tok = eouts[k - 2] if k >= 2 else senders
        gath = _gather(nodes, sr[:, k], tok)
        big, eout_chunk = _edge_mlp(
            k, big, gath, edges,
            We1[:H], We1[H:2 * H], We1[2 * H:], r1(be1),
            We2, r1(be2), We3, r1(be3), r1(ge), r1(bge))
        eouts.append(eout_chunk)
        accs.append(_scatter(eout_chunk, recv4[k], zeros))

    nodes_out = _node_mlp(
        nodes, accs,
        Wn1[:H], Wn1[H:], r1(bn1),
        Wn2, r1(bn2), Wn3, r1(bn3), r1(gn), r1(bgn))
    return (nodes_out, big)


# speedup vs baseline: 4.2766x; 1.0197x over previous
import functools

import jax
import jax.numpy as jnp
from jax import lax
from jax.experimental import pallas as pl
from jax.experimental.pallas import tpu as pltpu
from jax.experimental.pallas import tpu_sc as plsc

N, E, H = 10000, 320000, 128
NC, NS = 2, 16
NW = NC * NS
K = 5
EK = E // K
NBUF = 5

GPW = EK // NW
CHG = 80
NCHG = GPW // CHG
CHS = 40
NCHS = GPW // CHS


@functools.cache
def _mesh():
    return plsc.VectorSubcoreMesh(core_axis_name="c", subcore_axis_name="s",
                                  num_cores=NC, num_subcores=NS)


def _worker_id():
    return lax.axis_index("c") * NS + lax.axis_index("s")


def _gather_body(nodes_hbm, sr_hbm, tok_hbm, gout, idx_all, rows, gsems, wsems):
    del tok_hbm

    wid = _worker_id()
    ebase = wid * GPW

    for k in range(2):
        pltpu.sync_copy(sr_hbm.at[k, wid], idx_all.at[k])

    def chunk_refs(j):
        sel = j // NCHG
        r = j % NCHG
        idx = idx_all.at[sel, r]
        out = gout.at[sel, pl.ds(ebase + r * CHG, CHG)]
        return idx, out

    for b in range(NBUF):
        idx, _ = chunk_refs(b)
        pltpu.async_copy(nodes_hbm.at[idx], rows[b], gsems[b])

    def group(jj, carry):
        for b in range(NBUF):
            j = jj * NBUF + b
            idx, out = chunk_refs(j)
            pltpu.make_async_copy(nodes_hbm.at[idx], rows[b], gsems[b]).wait()
            pltpu.async_copy(rows[b], out, wsems[b])
        for b in range(NBUF):
            j2 = (jj + 1) * NBUF + b
            @pl.when(j2 < 2 * NCHG)
            def _():
                idx2, out2 = chunk_refs(j2)
                pltpu.make_async_copy(rows[b], out2, wsems[b]).wait()
                pltpu.async_copy(nodes_hbm.at[idx2], rows[b], gsems[b])
        return carry

    lax.fori_loop(0, (2 * NCHG) // NBUF, group, 0)

    for b in range(NBUF):
        pltpu.make_async_copy(
            rows[b], gout.at[0, pl.ds(ebase, CHG)], wsems[b]).wait()


@jax.jit
def _gather(nodes, sr3d, tok):
    return pl.kernel(
        _gather_body,
        out_type=jax.ShapeDtypeStruct((2, EK, H), jnp.float32),
        mesh=_mesh(),
        scratch_types=[
            pltpu.VMEM((2, NCHG, CHG), jnp.int32),
            [pltpu.VMEM((CHG, H), jnp.float32) for _ in range(NBUF)],
            [pltpu.SemaphoreType.DMA for _ in range(NBUF)],
            [pltpu.SemaphoreType.DMA for _ in range(NBUF)],
        ],
    )(nodes, sr3d, tok)


def _scatter_body(eout_hbm, recv_hbm, zeros_hbm, acc_out,
                  idx_all, rows, lsems, eff_sh):
    wid = _worker_id()
    c = lax.axis_index("c")
    s = lax.axis_index("s")
    ebase = wid * GPW
    rpw = 624
    tail_off = rpw * NS
    tail = N - tail_off

    pltpu.sync_copy(recv_hbm.at[wid], idx_all)

    pltpu.sync_copy(zeros_hbm.at[pl.ds(s * rpw, rpw)],
                    eff_sh.at[pl.ds(s * rpw, rpw)])
    @pl.when(s == 0)
    def _():
        pltpu.sync_copy(zeros_hbm.at[pl.ds(tail_off, tail)],
                        eff_sh.at[pl.ds(tail_off, tail)])
    plsc.subcore_barrier()

    def load_ref(j):
        return eout_hbm.at[pl.ds(ebase + j * CHS, CHS)]

    for b in range(NBUF):
        pltpu.async_copy(load_ref(b), rows[b], lsems[b])

    def group(jj, carry):
        for b in range(NBUF):
            j = jj * NBUF + b
            pltpu.make_async_copy(load_ref(j), rows[b], lsems[b]).wait()
            pltpu.sync_copy(rows[b], eff_sh.at[idx_all.at[j]], add=True)
            j2 = j + NBUF
            @pl.when(j2 < NCHS)
            def _():
                pltpu.async_copy(load_ref(j2), rows[b], lsems[b])
        return carry

    lax.fori_loop(0, NCHS // NBUF, group, 0)
    plsc.subcore_barrier()

    pltpu.sync_copy(eff_sh.at[pl.ds(s * rpw, rpw)],
                    acc_out.at[c, pl.ds(s * rpw, rpw)])
    @pl.when(s == 0)
    def _():
        pltpu.sync_copy(eff_sh.at[pl.ds(tail_off, tail)],
                        acc_out.at[c, pl.ds(tail_off, tail)])


@jax.jit
def _scatter(eout_chunk, recv3d, zeros):
    return pl.kernel(
        _scatter_body,
        out_type=jax.ShapeDtypeStruct((NC, N, H), jnp.float32),
        mesh=_mesh(),
        scratch_types=[
            pltpu.VMEM((NCHS, CHS), jnp.int32),
            [pltpu.VMEM((CHS, H), jnp.float32) for _ in range(NBUF)],
            [pltpu.SemaphoreType.DMA for _ in range(NBUF)],
            pltpu.VMEM_SHARED((N, H), jnp.float32),
        ],
    )(eout_chunk, recv3d, zeros)


def _edge_mlp_body(gath_src, gath_dst, edg_ref, wa_ref, wb_ref,
                   wc_ref, b1_ref, w2_ref, b2_ref, w3_ref, b3_ref, g_ref,
                   bg_ref, big_out, chunk_out):
    f32 = jnp.float32
    bf = lambda x: x.astype(jnp.bfloat16)
    h = jnp.dot(bf(gath_dst[0]), bf(wa_ref[...]), preferred_element_type=f32)
    h += jnp.dot(bf(gath_src[0]), bf(wb_ref[...]), preferred_element_type=f32)
    h += jnp.dot(bf(edg_ref[...]), bf(wc_ref[...]), preferred_element_type=f32)
    h = jnp.maximum(h + b1_ref[...], 0.0)
    h = jnp.maximum(
        jnp.dot(bf(h), bf(w2_ref[...]), preferred_element_type=f32)
        + b2_ref[...], 0.0)
    h = jnp.dot(bf(h), bf(w3_ref[...]), preferred_element_type=f32) + b3_ref[...]
    mu = jnp.mean(h, axis=-1, keepdims=True)
    d = h - mu
    var = jnp.mean(d * d, axis=-1, keepdims=True)
    out = d * lax.rsqrt(var + 1e-5) * g_ref[...] + bg_ref[...]
    big_out[...] = out
    chunk_out[...] = out


BE = 3200
BPC = EK // BE


def _edge_mlp(k, big, gath, edg, wa, wb, wc, b1, w2, b2, w3, b3, g, bg):
    src_spec = pl.BlockSpec((1, BE, H), lambda i: (0, i, 0))
    dst_spec = pl.BlockSpec((1, BE, H), lambda i: (1, i, 0))
    edg_spec = pl.BlockSpec((BE, H), lambda i, _k=k: (_k * BPC + i, 0))
    big_spec = pl.BlockSpec(memory_space=pltpu.MemorySpace.HBM)
    w_spec = pl.BlockSpec((H, H), lambda i: (0, 0))
    v_spec = pl.BlockSpec((1, H), lambda i: (0, 0))
    body = _edge_mlp_body
    in_specs = [src_spec, dst_spec, edg_spec,
                w_spec, w_spec, w_spec, v_spec,
                w_spec, v_spec, w_spec, v_spec, v_spec, v_spec]
    args = (gath, gath, edg, wa, wb, wc, b1, w2, b2, w3, b3, g, bg)
    aliases = {}
    if k > 0:
        body = lambda big_ref, *rest: _edge_mlp_body(*rest)
        in_specs = [big_spec] + in_specs
        args = (big,) + args
        aliases = {0: 0}
    return pl.pallas_call(
        body,
        grid=(BPC,),
        in_specs=in_specs,
        out_specs=[pl.BlockSpec((BE, H), lambda i, _k=k: (_k * BPC + i, 0)),
                   pl.BlockSpec((BE, H), lambda i: (i, 0))],
        out_shape=[jax.ShapeDtypeStruct((E, H), jnp.float32),
                   jax.ShapeDtypeStruct((EK, H), jnp.float32)],
        input_output_aliases=aliases,
    )(*args)


def _node_mlp_body(nod_ref, a0, a1, a2, a3, a4, wa_ref, wb_ref,
                   b1_ref, w2_ref, b2_ref, w3_ref, b3_ref, g_ref, bg_ref,
                   out_ref):
    f32 = jnp.float32
    eff = a0[0] + a0[1]
    for a in (a1, a2, a3, a4):
        eff += a[0] + a[1]
    h = jnp.dot(nod_ref[...], wa_ref[...], preferred_element_type=f32)
    h += jnp.dot(eff, wb_ref[...], preferred_element_type=f32)
    h = jnp.maximum(h + b1_ref[...], 0.0)
    h = jnp.maximum(
        jnp.dot(h, w2_ref[...], preferred_element_type=f32) + b2_ref[...], 0.0)
    h = jnp.dot(h, w3_ref[...], preferred_element_type=f32) + b3_ref[...]
    mu = jnp.mean(h, axis=-1, keepdims=True)
    d = h - mu
    var = jnp.mean(d * d, axis=-1, keepdims=True)
    out_ref[...] = d * lax.rsqrt(var + 1e-5) * g_ref[...] + bg_ref[...]


BN = 1000


def _node_mlp(nodes, accs, wa, wb, b1, w2, b2, w3, b3, g, bg):
    row_spec = pl.BlockSpec((BN, H), lambda i: (i, 0))
    acc_spec = pl.BlockSpec((NC, BN, H), lambda i: (0, i, 0))
    w_spec = pl.BlockSpec((H, H), lambda i: (0, 0))
    v_spec = pl.BlockSpec((1, H), lambda i: (0, 0))
    return pl.pallas_call(
        _node_mlp_body,
        grid=(N // BN,),
        in_specs=[row_spec] + [acc_spec] * K +
                 [w_spec, w_spec, v_spec,
                  w_spec, v_spec, w_spec, v_spec, v_spec, v_spec],
        out_specs=row_spec,
        out_shape=jax.ShapeDtypeStruct((N, H), jnp.float32),
    )(nodes, *accs, wa, wb, b1, w2, b2, w3, b3, g, bg)


def kernel(nodes, edges, senders, receivers, We1, be1, We2, be2, We3, be3,
           ge, bge, Wn1, bn1, Wn2, bn2, Wn3, bn3, gn, bgn):
    senders = senders.astype(jnp.int32)
    receivers = receivers.astype(jnp.int32)
    r1 = lambda v: v.reshape(1, H)

    sr = jnp.stack([senders, receivers]).reshape(2, K, NW, NCHG, CHG)
    recv4 = receivers.reshape(K, NW, NCHS, CHS)
    zeros = jnp.zeros((N, H), jnp.float32)
    big = None

    accs = []
    eouts = []
    for k in range(K):
        tok = eouts[k - 2] if k >= 2 else senders
        gath = _gather(nodes, sr[:, k], tok)
        big, eout_chunk = _edge_mlp(
            k, big, gath, edges,
            We1[:H], We1[H:2 * H], We1[2 * H:], r1(be1),
            We2, r1(be2), We3, r1(be3), r1(ge), r1(bge))
        eouts.append(eout_chunk)
        accs.append(_scatter(eout_chunk, recv4[k], zeros))

    nodes_out = _node_mlp(
        nodes, accs,
        Wn1[:H], Wn1[H:], r1(bn1),
        Wn2, r1(bn2), Wn3, r1(bn3), r1(gn), r1(bgn))
    return (nodes_out, big)


# speedup vs baseline: 4.2864x; 1.0023x over previous
import functools

import jax
import jax.numpy as jnp
from jax import lax
from jax.experimental import pallas as pl
from jax.experimental.pallas import tpu as pltpu
from jax.experimental.pallas import tpu_sc as plsc

N, E, H = 10000, 320000, 128
NC, NS = 2, 16
NW = NC * NS
K = 5
EK = E // K
NBUF = 5

GPW = EK // NW
CHG = 80
NCHG = GPW // CHG
CHS = 40
NCHS = GPW // CHS


@functools.cache
def _mesh():
    return plsc.VectorSubcoreMesh(core_axis_name="c", subcore_axis_name="s",
                                  num_cores=NC, num_subcores=NS)


def _worker_id():
    return lax.axis_index("c") * NS + lax.axis_index("s")


def _gather_body(nodes_hbm, sr_hbm, tok_hbm, gout, idx_all, rows, gsems, wsems):
    del tok_hbm

    wid = _worker_id()
    ebase = wid * GPW

    for k in range(2):
        pltpu.sync_copy(sr_hbm.at[k, wid], idx_all.at[k])

    def chunk_refs(j):
        sel = j // NCHG
        r = j % NCHG
        idx = idx_all.at[sel, r]
        out = gout.at[sel, pl.ds(ebase + r * CHG, CHG)]
        return idx, out

    for b in range(NBUF):
        idx, _ = chunk_refs(b)
        pltpu.async_copy(nodes_hbm.at[idx], rows[b], gsems[b])

    def group(jj, carry):
        for b in range(NBUF):
            j = jj * NBUF + b
            idx, out = chunk_refs(j)
            pltpu.make_async_copy(nodes_hbm.at[idx], rows[b], gsems[b]).wait()
            pltpu.async_copy(rows[b], out, wsems[b])
        for b in range(NBUF):
            j2 = (jj + 1) * NBUF + b
            @pl.when(j2 < 2 * NCHG)
            def _():
                idx2, out2 = chunk_refs(j2)
                pltpu.make_async_copy(rows[b], out2, wsems[b]).wait()
                pltpu.async_copy(nodes_hbm.at[idx2], rows[b], gsems[b])
        return carry

    lax.fori_loop(0, (2 * NCHG) // NBUF, group, 0)

    for b in range(NBUF):
        pltpu.make_async_copy(
            rows[b], gout.at[0, pl.ds(ebase, CHG)], wsems[b]).wait()


@jax.jit
def _gather(nodes, sr3d, tok):
    return pl.kernel(
        _gather_body,
        out_type=jax.ShapeDtypeStruct((2, EK, H), jnp.float32),
        mesh=_mesh(),
        scratch_types=[
            pltpu.VMEM((2, NCHG, CHG), jnp.int32),
            [pltpu.VMEM((CHG, H), jnp.float32) for _ in range(NBUF)],
            [pltpu.SemaphoreType.DMA for _ in range(NBUF)],
            [pltpu.SemaphoreType.DMA for _ in range(NBUF)],
        ],
    )(nodes, sr3d, tok)


def _scatter_body(eout_hbm, recv_hbm, zeros_hbm, acc_out,
                  idx_all, rows, lsems, eff_sh):
    wid = _worker_id()
    c = lax.axis_index("c")
    s = lax.axis_index("s")
    ebase = wid * GPW
    rpw = 624
    tail_off = rpw * NS
    tail = N - tail_off

    pltpu.sync_copy(recv_hbm.at[wid], idx_all)

    pltpu.sync_copy(zeros_hbm.at[pl.ds(s * rpw, rpw)],
                    eff_sh.at[pl.ds(s * rpw, rpw)])
    @pl.when(s == 0)
    def _():
        pltpu.sync_copy(zeros_hbm.at[pl.ds(tail_off, tail)],
                        eff_sh.at[pl.ds(tail_off, tail)])
    plsc.subcore_barrier()

    def load_ref(j):
        return eout_hbm.at[pl.ds(ebase + j * CHS, CHS)]

    for b in range(NBUF):
        pltpu.async_copy(load_ref(b), rows[b], lsems[b])

    def group(jj, carry):
        for b in range(NBUF):
            j = jj * NBUF + b
            pltpu.make_async_copy(load_ref(j), rows[b], lsems[b]).wait()
            pltpu.sync_copy(rows[b], eff_sh.at[idx_all.at[j]], add=True)
            j2 = j + NBUF
            @pl.when(j2 < NCHS)
            def _():
                pltpu.async_copy(load_ref(j2), rows[b], lsems[b])
        return carry

    lax.fori_loop(0, NCHS // NBUF, group, 0)
    plsc.subcore_barrier()

    pltpu.sync_copy(eff_sh.at[pl.ds(s * rpw, rpw)],
                    acc_out.at[c, pl.ds(s * rpw, rpw)])
    @pl.when(s == 0)
    def _():
        pltpu.sync_copy(eff_sh.at[pl.ds(tail_off, tail)],
                        acc_out.at[c, pl.ds(tail_off, tail)])


@jax.jit
def _scatter(eout_chunk, recv3d, zeros):
    return pl.kernel(
        _scatter_body,
        out_type=jax.ShapeDtypeStruct((NC, N, H), jnp.float32),
        mesh=_mesh(),
        scratch_types=[
            pltpu.VMEM((NCHS, CHS), jnp.int32),
            [pltpu.VMEM((CHS, H), jnp.float32) for _ in range(NBUF)],
            [pltpu.SemaphoreType.DMA for _ in range(NBUF)],
            pltpu.VMEM_SHARED((N, H), jnp.float32),
        ],
    )(eout_chunk, recv3d, zeros)


def _edge_mlp_body(gath_src, gath_dst, edg_ref, wa_ref, wb_ref,
                   wc_ref, b1_ref, w2_ref, b2_ref, w3_ref, b3_ref, g_ref,
                   bg_ref, big_out, chunk_out):
    f32 = jnp.float32
    bf = lambda x: x.astype(jnp.bfloat16)
    h = jnp.dot(bf(gath_dst[0]), bf(wa_ref[...]), preferred_element_type=f32)
    h += jnp.dot(bf(gath_src[0]), bf(wb_ref[...]), preferred_element_type=f32)
    h += jnp.dot(bf(edg_ref[...]), bf(wc_ref[...]), preferred_element_type=f32)
    h = jnp.maximum(h + b1_ref[...], 0.0)
    h = jnp.maximum(
        jnp.dot(bf(h), bf(w2_ref[...]), preferred_element_type=f32)
        + b2_ref[...], 0.0)
    h = jnp.dot(bf(h), bf(w3_ref[...]), preferred_element_type=f32) + b3_ref[...]
    mu = jnp.mean(h, axis=-1, keepdims=True)
    d = h - mu
    var = jnp.mean(d * d, axis=-1, keepdims=True)
    out = d * lax.rsqrt(var + 1e-5) * g_ref[...] + bg_ref[...]
    big_out[...] = out
    chunk_out[...] = out


BE = 6400
BPC = EK // BE


def _edge_mlp(k, big, gath, edg, wa, wb, wc, b1, w2, b2, w3, b3, g, bg):
    src_spec = pl.BlockSpec((1, BE, H), lambda i: (0, i, 0))
    dst_spec = pl.BlockSpec((1, BE, H), lambda i: (1, i, 0))
    edg_spec = pl.BlockSpec((BE, H), lambda i, _k=k: (_k * BPC + i, 0))
    big_spec = pl.BlockSpec(memory_space=pltpu.MemorySpace.HBM)
    w_spec = pl.BlockSpec((H, H), lambda i: (0, 0))
    v_spec = pl.BlockSpec((1, H), lambda i: (0, 0))
    body = _edge_mlp_body
    in_specs = [src_spec, dst_spec, edg_spec,
                w_spec, w_spec, w_spec, v_spec,
                w_spec, v_spec, w_spec, v_spec, v_spec, v_spec]
    args = (gath, gath, edg, wa, wb, wc, b1, w2, b2, w3, b3, g, bg)
    aliases = {}
    if k > 0:
        body = lambda big_ref, *rest: _edge_mlp_body(*rest)
        in_specs = [big_spec] + in_specs
        args = (big,) + args
        aliases = {0: 0}
    return pl.pallas_call(
        body,
        grid=(BPC,),
        in_specs=in_specs,
        out_specs=[pl.BlockSpec((BE, H), lambda i, _k=k: (_k * BPC + i, 0)),
                   pl.BlockSpec((BE, H), lambda i: (i, 0))],
        out_shape=[jax.ShapeDtypeStruct((E, H), jnp.float32),
                   jax.ShapeDtypeStruct((EK, H), jnp.float32)],
        input_output_aliases=aliases,
    )(*args)


def _node_mlp_body(nod_ref, a0, a1, a2, a3, a4, wa_ref, wb_ref,
                   b1_ref, w2_ref, b2_ref, w3_ref, b3_ref, g_ref, bg_ref,
                   out_ref):
    f32 = jnp.float32
    eff = a0[0] + a0[1]
    for a in (a1, a2, a3, a4):
        eff += a[0] + a[1]
    h = jnp.dot(nod_ref[...], wa_ref[...], preferred_element_type=f32)
    h += jnp.dot(eff, wb_ref[...], preferred_element_type=f32)
    h = jnp.maximum(h + b1_ref[...], 0.0)
    h = jnp.maximum(
        jnp.dot(h, w2_ref[...], preferred_element_type=f32) + b2_ref[...], 0.0)
    h = jnp.dot(h, w3_ref[...], preferred_element_type=f32) + b3_ref[...]
    mu = jnp.mean(h, axis=-1, keepdims=True)
    d = h - mu
    var = jnp.mean(d * d, axis=-1, keepdims=True)
    out_ref[...] = d * lax.rsqrt(var + 1e-5) * g_ref[...] + bg_ref[...]


BN = 1000


def _node_mlp(nodes, accs, wa, wb, b1, w2, b2, w3, b3, g, bg):
    row_spec = pl.BlockSpec((BN, H), lambda i: (i, 0))
    acc_spec = pl.BlockSpec((NC, BN, H), lambda i: (0, i, 0))
    w_spec = pl.BlockSpec((H, H), lambda i: (0, 0))
    v_spec = pl.BlockSpec((1, H), lambda i: (0, 0))
    return pl.pallas_call(
        _node_mlp_body,
        grid=(N // BN,),
        in_specs=[row_spec] + [acc_spec] * K +
                 [w_spec, w_spec, v_spec,
                  w_spec, v_spec, w_spec, v_spec, v_spec, v_spec],
        out_specs=row_spec,
        out_shape=jax.ShapeDtypeStruct((N, H), jnp.float32),
    )(nodes, *accs, wa, wb, b1, w2, b2, w3, b3, g, bg)


def kernel(nodes, edges, senders, receivers, We1, be1, We2, be2, We3, be3,
           ge, bge, Wn1, bn1, Wn2, bn2, Wn3, bn3, gn, bgn):
    senders = senders.astype(jnp.int32)
    receivers = receivers.astype(jnp.int32)
    r1 = lambda v: v.reshape(1, H)

    sr = jnp.stack([senders, receivers]).reshape(2, K, NW, NCHG, CHG)
    recv4 = receivers.reshape(K, NW, NCHS, CHS)
    zeros = jnp.zeros((N, H), jnp.float32)
    big = None

    accs = []
    eouts = []
    for k in range(K):
        tok = eouts[k - 2] if k >= 2 else senders
        gath = _gather(nodes, sr[:, k], tok)
        big, eout_chunk = _edge_mlp(
            k, big, gath, edges,
            We1[:H], We1[H:2 * H], We1[2 * H:], r1(be1),
            We2, r1(be2), We3, r1(be3), r1(ge), r1(bge))
        eouts.append(eout_chunk)
        accs.append(_scatter(eout_chunk, recv4[k], zeros))

    nodes_out = _node_mlp(
        nodes, accs,
        Wn1[:H], Wn1[H:], r1(bn1),
        Wn2, r1(bn2), Wn3, r1(bn3), r1(gn), r1(bgn))
    return (nodes_out, big)
